# Initial kernel scaffold; baseline (speedup 1.0000x reference)
import jax
import jax.numpy as jnp
from jax.experimental import pallas as pl


def kernel(x, edge_index, ud_edges, params):
    raise NotImplementedError("write your pallas kernel here")


# speedup vs baseline: 13.9296x; 13.9296x over previous
import functools

import jax
import jax.numpy as jnp
from jax import lax
from jax.experimental import pallas as pl
from jax.experimental.pallas import tpu as pltpu
from jax.experimental.pallas import tpu_sc as plsc

N = 100000
N_PAD = 100352
PAD_ROWS = N_PAD - N
E_PAD = 1605632
M_PAD = 819200
NC, NS = 2, 16
K = 8
CHUNK = K * 128
ROWS_PER_SUB = N_PAD // NS
SLOPE = 0.1

@functools.lru_cache(maxsize=None)
def _mesh():
  return plsc.VectorSubcoreMesh(
      core_axis_name="c", subcore_axis_name="s", num_cores=NC, num_subcores=NS)


def _leaky(h):
  return jnp.where(h > 0, h, SLOPE * h)


@functools.lru_cache(maxsize=None)
def _sc_deg():
  return pl.kernel(
      _sc_deg_body,
      out_type=jax.ShapeDtypeStruct((NC, N_PAD), jnp.float32),
      mesh=_mesh(),
      compiler_params=pltpu.CompilerParams(use_tc_tiling_on_sc=False),
      scratch_types=[
          pltpu.VMEM((K, 128), jnp.int32),
          pltpu.VMEM((128,), jnp.float32),
          pltpu.VMEM_SHARED((N_PAD,), jnp.float32),
      ],
  )


def _sc_deg_body(dst2_hbm, zeros1_hbm, out_hbm, dst_v, ones_v, acc):
  c = lax.axis_index("c")
  s = lax.axis_index("s")
  wid = c * NS + s
  pltpu.sync_copy(zeros1_hbm, acc.at[pl.ds(s * ROWS_PER_SUB, ROWS_PER_SUB)])
  for i in range(K):
    ones_v[pl.ds(i * 16, 16)] = jnp.ones((16,), jnp.float32)
  plsc.subcore_barrier()
  idx_rows = E_PAD // 32 // 128
  row0 = wid * idx_rows

  def chunk(i, carry):
    r = row0 + i * K
    pltpu.sync_copy(dst2_hbm.at[pl.ds(r, K)], dst_v)
    for j in range(K):
      pltpu.sync_copy(ones_v, acc.at[dst_v.at[j]], add=True)
    return carry

  lax.fori_loop(0, idx_rows // K, chunk, 0)
  plsc.subcore_barrier()
  sl = pl.ds(s * ROWS_PER_SUB, ROWS_PER_SUB)
  pltpu.sync_copy(acc.at[sl], out_hbm.at[c, sl])


@functools.lru_cache(maxsize=None)
def _make_agg_edge_split(width):
  def agg(src2_hbm, dst2_hbm, y_hbm, zeros_hbm, out_hbm,
          src_v, dst_v, rows_v, acc, sem):
    c = lax.axis_index("c")
    s = lax.axis_index("s")
    wid = c * NS + s
    pltpu.sync_copy(zeros_hbm, acc.at[pl.ds(s * ROWS_PER_SUB, ROWS_PER_SUB)])
    plsc.subcore_barrier()
    idx_rows = E_PAD // 32 // 128
    row0 = wid * idx_rows

    def chunk(i, carry):
      r = row0 + i * K
      pltpu.sync_copy(src2_hbm.at[pl.ds(r, K)], src_v)
      pltpu.sync_copy(dst2_hbm.at[pl.ds(r, K)], dst_v)
      descs = [
          pltpu.async_copy(y_hbm.at[src_v.at[j]],
                           rows_v.at[pl.ds(j * 128, 128)], sem)
          for j in range(K)
      ]
      for d in descs:
        d.wait()
      for j in range(K):
        pltpu.sync_copy(rows_v.at[pl.ds(j * 128, 128)],
                        acc.at[dst_v.at[j]], add=True)
      return carry

    lax.fori_loop(0, idx_rows // K, chunk, 0)
    plsc.subcore_barrier()
    sl = pl.ds(s * ROWS_PER_SUB, ROWS_PER_SUB)
    pltpu.sync_copy(acc.at[sl], out_hbm.at[c, sl])

  return pl.kernel(
      agg,
      out_type=jax.ShapeDtypeStruct((NC, N_PAD, width), jnp.float32),
      mesh=_mesh(),
      compiler_params=pltpu.CompilerParams(use_tc_tiling_on_sc=False),
      scratch_types=[
          pltpu.VMEM((K, 128), jnp.int32),
          pltpu.VMEM((K, 128), jnp.int32),
          pltpu.VMEM((CHUNK, width), jnp.float32),
          pltpu.VMEM_SHARED((N_PAD, width), jnp.float32),
          pltpu.SemaphoreType.DMA,
      ],
  )


@functools.lru_cache(maxsize=None)
def _sc_agg_feat_split():
  return pl.kernel(
      _sc_agg_feat_split_body,
      out_type=jax.ShapeDtypeStruct((NC, N_PAD, 16), jnp.float32),
      mesh=_mesh(),
      compiler_params=pltpu.CompilerParams(use_tc_tiling_on_sc=False),
      scratch_types=[
          pltpu.VMEM((K, 128), jnp.int32),
          pltpu.VMEM((K, 128), jnp.int32),
          pltpu.VMEM((CHUNK, 16), jnp.float32),
          pltpu.VMEM_SHARED((N_PAD, 16), jnp.float32),
          pltpu.SemaphoreType.DMA,
      ],
  )


def _sc_agg_feat_split_body(src2_hbm, dst2_hbm, ytab_hbm, zeros_hbm, out_hbm,
                            src_v, dst_v, rows_v, acc, sem):
  c = lax.axis_index("c")
  s = lax.axis_index("s")
  bias = c * N_PAD
  pltpu.sync_copy(zeros_hbm, acc.at[pl.ds(s * ROWS_PER_SUB, ROWS_PER_SUB)])
  plsc.subcore_barrier()
  idx_rows = E_PAD // NS // 128
  row0 = s * idx_rows

  def chunk(i, carry):
    r = row0 + i * K
    pltpu.sync_copy(src2_hbm.at[pl.ds(r, K)], src_v)
    pltpu.sync_copy(dst2_hbm.at[pl.ds(r, K)], dst_v)
    for j in range(K):
      for t in range(8):
        sl = (j, pl.ds(t * 16, 16))
        src_v[sl] = src_v[sl] + bias
    descs = [
        pltpu.async_copy(ytab_hbm.at[src_v.at[j]],
                         rows_v.at[pl.ds(j * 128, 128)], sem)
        for j in range(K)
    ]
    for d in descs:
      d.wait()
    for j in range(K):
      pltpu.sync_copy(rows_v.at[pl.ds(j * 128, 128)],
                      acc.at[dst_v.at[j]], add=True)
    return carry

  lax.fori_loop(0, idx_rows // K, chunk, 0)
  plsc.subcore_barrier()
  sl = pl.ds(s * ROWS_PER_SUB, ROWS_PER_SUB)
  pltpu.sync_copy(acc.at[sl], out_hbm.at[c, sl])


@functools.lru_cache(maxsize=None)
def _sc_edge_feat():
  return pl.kernel(
      _sc_edge_feat_body,
      out_type=jax.ShapeDtypeStruct((M_PAD, 64), jnp.float32),
      mesh=_mesh(),
      compiler_params=pltpu.CompilerParams(use_tc_tiling_on_sc=False),
      scratch_types=[
          pltpu.VMEM((K, 128), jnp.int32),
          pltpu.VMEM((K, 128), jnp.int32),
          pltpu.VMEM((CHUNK, 64), jnp.float32),
          pltpu.SemaphoreType.DMA,
      ],
  )


def _sc_edge_feat_body(ud0_hbm, ud1_hbm, x_hbm, out_hbm, i0_v, i1_v, rows_v, sem):
  c = lax.axis_index("c")
  s = lax.axis_index("s")
  wid = c * NS + s
  idx_rows = M_PAD // 32 // 128
  row0 = wid * idx_rows

  def chunk(i, carry):
    r = row0 + i * K
    pltpu.sync_copy(ud0_hbm.at[pl.ds(r, K)], i0_v)
    pltpu.sync_copy(ud1_hbm.at[pl.ds(r, K)], i1_v)
    descs = [
        pltpu.async_copy(x_hbm.at[i0_v.at[j]],
                         rows_v.at[pl.ds(j * 128, 128)], sem)
        for j in range(K)
    ]
    for d in descs:
      d.wait()
    descs = [
        pltpu.async_copy(x_hbm.at[i1_v.at[j]],
                         rows_v.at[pl.ds(j * 128, 128)], sem, add=True)
        for j in range(K)
    ]
    for d in descs:
      d.wait()
    pltpu.sync_copy(rows_v, out_hbm.at[pl.ds(r * 128, CHUNK)])
    return carry

  lax.fori_loop(0, idx_rows // K, chunk, 0)


_R = 2048


def _dinv_of(deg_ref):
  d = deg_ref[0, :] + deg_ref[1, :]
  return jnp.where(d > 0, lax.rsqrt(d), 0.0)


def _tc_prescale_body(deg_ref, x_ref, y_ref):
  dinv = _dinv_of(deg_ref)
  y_ref[...] = x_ref[...] * dinv[:, None]


def _tc_conv1_body(deg_ref, agg_ref, w_ref, b_ref, x1_ref, y2_ref):
  dinv = _dinv_of(deg_ref)
  agg = (agg_ref[0] + agg_ref[1]) * dinv[:, None]
  h = jnp.dot(agg, w_ref[...], preferred_element_type=jnp.float32) + b_ref[...]
  x1 = _leaky(h)
  x1_ref[...] = x1
  y2 = x1 * dinv[:, None]
  y2_ref[0] = y2[:, :16]
  y2_ref[1] = y2[:, 16:]


def _tc_conv2_body(deg_ref, agg_ref, x1_ref, w_ref, b_ref, x2_ref, y3_ref):
  dinv = _dinv_of(deg_ref)
  agg = jnp.concatenate([agg_ref[0], agg_ref[1]], axis=1) * dinv[:, None]
  h = jnp.dot(agg, w_ref[...], preferred_element_type=jnp.float32) + b_ref[...]
  x2 = _leaky(h) + x1_ref[...]
  x2_ref[...] = x2
  y3 = x2 * dinv[:, None]
  y3_ref[0] = y3[:, :16]
  y3_ref[1] = y3[:, 16:]


def _tc_conv3_body(deg_ref, agg_ref, w_ref, b_ref, x3_ref):
  dinv = _dinv_of(deg_ref)
  agg = jnp.concatenate([agg_ref[0], agg_ref[1]], axis=1) * dinv[:, None]
  x3_ref[...] = (
      jnp.dot(agg, w_ref[...], preferred_element_type=jnp.float32) + b_ref[...])


def _mlp3(h, w0, b0, w1, b1, w2, b2):
  h = _leaky(jnp.dot(h, w0, preferred_element_type=jnp.float32) + b0)
  h = _leaky(jnp.dot(h, w1, preferred_element_type=jnp.float32) + b1)
  return jnp.dot(h, w2, preferred_element_type=jnp.float32) + b2


def _tc_heads_body(x_ref, aw0, ab0, aw1, ab1, aw2, ab2,
                   bw0, bb0, bw1, bb1, bw2, bb2, oa_ref, ob_ref):
  h = x_ref[...]
  oa_ref[...] = _mlp3(h, aw0[...], ab0[...], aw1[...], ab1[...], aw2[...], ab2[...])
  ob_ref[...] = _mlp3(h, bw0[...], bb0[...], bw1[...], bb1[...], bw2[...], bb2[...])


def _full(shape):
  zeros = (0,) * len(shape)
  return pl.BlockSpec(shape, lambda i, z=zeros: z)


def _rows(shape, dim=0):
  def imap(i):
    idx = [0] * len(shape)
    idx[dim] = i
    return tuple(idx)
  return pl.BlockSpec(shape, imap)


def _run_heads(xarr, rows_total, rblock, p, n1, n2, d1, d2):
  grid = rows_total // rblock
  wspecs = []
  wargs = []
  for nm, dout in ((n1, d1), (n2, d2)):
    for li, (di, do) in enumerate(((64, 64), (64, 64), (64, dout))):
      wargs += [p[f'{nm}_w{li}'], p[f'{nm}_b{li}'].reshape(1, do)]
      wspecs += [_full((di, do)), _full((1, do))]
  return pl.pallas_call(
      _tc_heads_body,
      grid=(grid,),
      in_specs=[_rows((rblock, 64))] + wspecs,
      out_specs=[_rows((rblock, d1)), _rows((rblock, d2))],
      out_shape=[jax.ShapeDtypeStruct((rows_total, d1), jnp.float32),
                 jax.ShapeDtypeStruct((rows_total, d2), jnp.float32)],
  )(xarr, *wargs)


def kernel(x, edge_index, ud_edges, params):
  p = params
  f32 = jnp.float32

  dummy = lambda n: (jnp.arange(n, dtype=jnp.int32) % PAD_ROWS) + N
  src = jnp.concatenate([edge_index[0].astype(jnp.int32),
                         dummy(E_PAD - edge_index.shape[1])])
  dst = jnp.concatenate([edge_index[1].astype(jnp.int32),
                         dummy(E_PAD - edge_index.shape[1])])
  src2 = src.reshape(E_PAD // 128, 128)
  dst2 = dst.reshape(E_PAD // 128, 128)
  m = ud_edges.shape[0]
  ud0 = jnp.concatenate([ud_edges[:, 0].astype(jnp.int32), dummy(M_PAD - m)])
  ud1 = jnp.concatenate([ud_edges[:, 1].astype(jnp.int32), dummy(M_PAD - m)])
  ud0_2 = ud0.reshape(M_PAD // 128, 128)
  ud1_2 = ud1.reshape(M_PAD // 128, 128)
  x8 = jnp.pad(x[:, :6].astype(f32), ((0, N_PAD - N), (0, 2)))
  w0p = jnp.pad(p['gcn_w0'], ((0, 2), (0, 0)))
  z1 = jnp.zeros((ROWS_PER_SUB,), f32)
  z8 = jnp.zeros((ROWS_PER_SUB, 8), f32)
  z16 = jnp.zeros((ROWS_PER_SUB, 16), f32)

  deg2 = _sc_deg()(dst2, z1)

  y1 = pl.pallas_call(
      _tc_prescale_body,
      grid=(N_PAD // _R,),
      in_specs=[_rows((2, _R), dim=1), _rows((_R, 8))],
      out_specs=_rows((_R, 8)),
      out_shape=jax.ShapeDtypeStruct((N_PAD, 8), f32),
  )(deg2, x8)
  agg1 = _make_agg_edge_split(8)(src2, dst2, y1, z8)
  x1, y2 = pl.pallas_call(
      _tc_conv1_body,
      grid=(N_PAD // _R,),
      in_specs=[_rows((2, _R), dim=1), _rows((2, _R, 8), dim=1),
                _full((8, 32)), _full((1, 32))],
      out_specs=[_rows((_R, 32)), _rows((2, _R, 16), dim=1)],
      out_shape=[jax.ShapeDtypeStruct((N_PAD, 32), f32),
                 jax.ShapeDtypeStruct((2, N_PAD, 16), f32)],
  )(deg2, agg1, w0p, p['gcn_b0'].reshape(1, 32))

  agg2 = _sc_agg_feat_split()(src2, dst2, y2.reshape(2 * N_PAD, 16), z16)
  x2, y3 = pl.pallas_call(
      _tc_conv2_body,
      grid=(N_PAD // _R,),
      in_specs=[_rows((2, _R), dim=1), _rows((2, _R, 16), dim=1),
                _rows((_R, 32)), _full((32, 32)), _full((1, 32))],
      out_specs=[_rows((_R, 32)), _rows((2, _R, 16), dim=1)],
      out_shape=[jax.ShapeDtypeStruct((N_PAD, 32), f32),
                 jax.ShapeDtypeStruct((2, N_PAD, 16), f32)],
  )(deg2, agg2, x1, p['gcn_w1'], p['gcn_b1'].reshape(1, 32))

  agg3 = _sc_agg_feat_split()(src2, dst2, y3.reshape(2 * N_PAD, 16), z16)
  x3 = pl.pallas_call(
      _tc_conv3_body,
      grid=(N_PAD // _R,),
      in_specs=[_rows((2, _R), dim=1), _rows((2, _R, 16), dim=1),
                _full((32, 64)), _full((1, 64))],
      out_specs=_rows((_R, 64)),
      out_shape=jax.ShapeDtypeStruct((N_PAD, 64), f32),
  )(deg2, agg3, p['gcn_w2'], p['gcn_b2'].reshape(1, 64))

  ef = _sc_edge_feat()(ud0_2, ud1_2, x3)

  vp, vd = _run_heads(x3, N, 2000, p, 'pn', 'dn', 10, 10)
  ep, ed = _run_heads(ef, m, 2000, p, 'pe', 'de', 10, 16)

  return (x3[:N], vp, vd, ep, ed)


# speedup vs baseline: 14.8877x; 1.0688x over previous
import functools

import jax
import jax.numpy as jnp
from jax import lax
from jax.experimental import pallas as pl
from jax.experimental.pallas import tpu as pltpu
from jax.experimental.pallas import tpu_sc as plsc

N = 100000
N_PAD = 100352
PAD_ROWS = N_PAD - N
E_PAD = 1605632
M_PAD = 819200
NC, NS = 2, 16
K = 8
CHUNK = K * 128
ROWS_PER_SUB = N_PAD // NS
SLOPE = 0.1

@functools.lru_cache(maxsize=None)
def _mesh():
  return plsc.VectorSubcoreMesh(
      core_axis_name="c", subcore_axis_name="s", num_cores=NC, num_subcores=NS)


def _leaky(h):
  return jnp.where(h > 0, h, SLOPE * h)


KD = 4


@functools.lru_cache(maxsize=None)
def _sc_deg():
  return pl.kernel(
      _sc_deg_body,
      out_type=jax.ShapeDtypeStruct((NC, N_PAD), jnp.float32),
      mesh=_mesh(),
      compiler_params=pltpu.CompilerParams(use_tc_tiling_on_sc=False),
      scratch_types=[
          pltpu.VMEM((2, KD, 128), jnp.int32),
          pltpu.VMEM((128,), jnp.float32),
          pltpu.VMEM_SHARED((N_PAD,), jnp.float32),
          pltpu.SemaphoreType.DMA((2,)),
      ],
  )


def _sc_deg_body(dst2_hbm, zeros1_hbm, out_hbm, dst_v, ones_v, acc, ssem):
  c = lax.axis_index("c")
  s = lax.axis_index("s")
  wid = c * NS + s
  pltpu.sync_copy(zeros1_hbm, acc.at[pl.ds(s * ROWS_PER_SUB, ROWS_PER_SUB)])
  for i in range(8):
    ones_v[pl.ds(i * 16, 16)] = jnp.ones((16,), jnp.float32)
  plsc.subcore_barrier()
  worker_rows = E_PAD // 32 // 128
  row0 = wid * worker_rows
  n_pairs = worker_rows // KD // 2
  SB = KD * 128 * 4

  def load(ci, b):
    pltpu.sync_copy(dst2_hbm.at[pl.ds(row0 + ci * KD, KD)], dst_v.at[b])

  def issue_s(b):
    for j in range(KD):
      pltpu.async_copy(ones_v, acc.at[dst_v.at[b, j]], ssem.at[b], add=True)

  def drain_s(b):
    for j in range(KD):
      pltpu.make_async_copy(ones_v, acc.at[dst_v.at[b, j]], ssem.at[b]).wait()

  load(0, 0)

  def pair(p, carry):
    issue_s(0)
    @pl.when(p > 0)
    def _():
      drain_s(1)
    load(2 * p + 1, 1)
    issue_s(1)
    @pl.when(p < n_pairs - 1)
    def _():
      drain_s(0)
      load(2 * p + 2, 0)
    return carry

  lax.fori_loop(0, n_pairs, pair, 0)
  drain_s(0)
  drain_s(1)
  plsc.subcore_barrier()
  sl = pl.ds(s * ROWS_PER_SUB, ROWS_PER_SUB)
  pltpu.sync_copy(acc.at[sl], out_hbm.at[c, sl])


def _agg_body(src2_hbm, dst2_hbm, ytab_hbm, zeros_hbm, out_hbm,
              src_v, dst_v, rows_v, acc, gsem, ssem, *, width, k, feat_split):
  c = lax.axis_index("c")
  s = lax.axis_index("s")
  pltpu.sync_copy(zeros_hbm, acc.at[pl.ds(s * ROWS_PER_SUB, ROWS_PER_SUB)])
  plsc.subcore_barrier()
  if feat_split:
    worker_rows = E_PAD // NS // 128
    row0 = s * worker_rows
    bias = c * N_PAD
  else:
    worker_rows = E_PAD // 32 // 128
    row0 = (c * NS + s) * worker_rows
  n_pairs = worker_rows // k // 2
  GB = k * 128 * width * 4

  def load(ci, b):
    r = row0 + ci * k
    pltpu.sync_copy(src2_hbm.at[pl.ds(r, k)], src_v.at[b])
    pltpu.sync_copy(dst2_hbm.at[pl.ds(r, k)], dst_v.at[b])
    if feat_split:
      for j in range(k):
        for t in range(8):
          sl = (b, j, pl.ds(t * 16, 16))
          src_v[sl] = src_v[sl] + bias

  def issue_g(b):
    for j in range(k):
      pltpu.async_copy(ytab_hbm.at[src_v.at[b, j]],
                       rows_v.at[b, pl.ds(j * 128, 128)], gsem.at[b])

  def issue_s(b):
    for j in range(k):
      pltpu.async_copy(rows_v.at[b, pl.ds(j * 128, 128)],
                       acc.at[dst_v.at[b, j]], ssem.at[b], add=True)

  def drain_g(b):
    for j in range(k):
      pltpu.make_async_copy(ytab_hbm.at[src_v.at[b, j]],
                            rows_v.at[b, pl.ds(j * 128, 128)],
                            gsem.at[b]).wait()

  def drain_s(b):
    for j in range(k):
      pltpu.make_async_copy(rows_v.at[b, pl.ds(j * 128, 128)],
                            acc.at[dst_v.at[b, j]], ssem.at[b]).wait()

  load(0, 0)
  issue_g(0)

  def pair(p, carry):
    @pl.when(p > 0)
    def _():
      drain_s(1)
    load(2 * p + 1, 1)
    issue_g(1)
    drain_g(0)
    issue_s(0)
    @pl.when(p < n_pairs - 1)
    def _():
      drain_s(0)
      load(2 * p + 2, 0)
      issue_g(0)
    drain_g(1)
    issue_s(1)
    return carry

  lax.fori_loop(0, n_pairs, pair, 0)
  drain_s(0)
  drain_s(1)
  plsc.subcore_barrier()
  sl = pl.ds(s * ROWS_PER_SUB, ROWS_PER_SUB)
  pltpu.sync_copy(acc.at[sl], out_hbm.at[c, sl])


@functools.lru_cache(maxsize=None)
def _make_agg_edge_split(width):
  k = KD
  body = functools.partial(_agg_body, width=width, k=k, feat_split=False)
  return pl.kernel(
      body,
      out_type=jax.ShapeDtypeStruct((NC, N_PAD, width), jnp.float32),
      mesh=_mesh(),
      compiler_params=pltpu.CompilerParams(use_tc_tiling_on_sc=False),
      scratch_types=[
          pltpu.VMEM((2, k, 128), jnp.int32),
          pltpu.VMEM((2, k, 128), jnp.int32),
          pltpu.VMEM((2, k * 128, width), jnp.float32),
          pltpu.VMEM_SHARED((N_PAD, width), jnp.float32),
          pltpu.SemaphoreType.DMA((2,)),
          pltpu.SemaphoreType.DMA((2,)),
      ],
  )


@functools.lru_cache(maxsize=None)
def _sc_agg_feat_split():
  k = KD
  body = functools.partial(_agg_body, width=16, k=k, feat_split=True)
  return pl.kernel(
      body,
      out_type=jax.ShapeDtypeStruct((NC, N_PAD, 16), jnp.float32),
      mesh=_mesh(),
      compiler_params=pltpu.CompilerParams(use_tc_tiling_on_sc=False),
      scratch_types=[
          pltpu.VMEM((2, k, 128), jnp.int32),
          pltpu.VMEM((2, k, 128), jnp.int32),
          pltpu.VMEM((2, k * 128, 16), jnp.float32),
          pltpu.VMEM_SHARED((N_PAD, 16), jnp.float32),
          pltpu.SemaphoreType.DMA((2,)),
          pltpu.SemaphoreType.DMA((2,)),
      ],
  )


@functools.lru_cache(maxsize=None)
def _sc_edge_feat():
  return pl.kernel(
      _sc_edge_feat_body,
      out_type=jax.ShapeDtypeStruct((M_PAD, 64), jnp.float32),
      mesh=_mesh(),
      compiler_params=pltpu.CompilerParams(use_tc_tiling_on_sc=False),
      scratch_types=[
          pltpu.VMEM((2, KD, 128), jnp.int32),
          pltpu.VMEM((2, KD, 128), jnp.int32),
          pltpu.VMEM((2, KD * 128, 64), jnp.float32),
          pltpu.SemaphoreType.DMA((2,)),
          pltpu.SemaphoreType.DMA((2,)),
      ],
  )


def _sc_edge_feat_body(ud0_hbm, ud1_hbm, x_hbm, out_hbm,
                       i0_v, i1_v, rows_v, gsem, osem):
  c = lax.axis_index("c")
  s = lax.axis_index("s")
  wid = c * NS + s
  k = KD
  worker_rows = M_PAD // 32 // 128
  row0 = wid * worker_rows
  n_pairs = worker_rows // k // 2
  GB = k * 128 * 64 * 4

  def load(ci, b):
    r = row0 + ci * k
    pltpu.sync_copy(ud0_hbm.at[pl.ds(r, k)], i0_v.at[b])
    pltpu.sync_copy(ud1_hbm.at[pl.ds(r, k)], i1_v.at[b])

  def g1(b):
    for j in range(k):
      pltpu.async_copy(x_hbm.at[i0_v.at[b, j]],
                       rows_v.at[b, pl.ds(j * 128, 128)], gsem.at[b])

  def g2(b):
    for j in range(k):
      pltpu.async_copy(x_hbm.at[i1_v.at[b, j]],
                       rows_v.at[b, pl.ds(j * 128, 128)], gsem.at[b], add=True)

  def store(ci, b):
    pltpu.async_copy(rows_v.at[b],
                     out_hbm.at[pl.ds((row0 + ci * k) * 128, k * 128)],
                     osem.at[b])

  def drain_g(b, iv):
    for j in range(k):
      pltpu.make_async_copy(x_hbm.at[iv.at[b, j]],
                            rows_v.at[b, pl.ds(j * 128, 128)],
                            gsem.at[b]).wait()

  def drain_o(b):
    pltpu.make_async_copy(rows_v.at[b], out_hbm.at[pl.ds(0, k * 128)],
                          osem.at[b]).wait()

  load(0, 0)
  g1(0)

  def pair(p, carry):
    @pl.when(p > 0)
    def _():
      drain_o(1)
    load(2 * p + 1, 1)
    g1(1)
    drain_g(0, i0_v)
    g2(0)
    drain_g(0, i1_v)
    store(2 * p, 0)
    @pl.when(p < n_pairs - 1)
    def _():
      drain_o(0)
      load(2 * p + 2, 0)
      g1(0)
    drain_g(1, i0_v)
    g2(1)
    drain_g(1, i1_v)
    store(2 * p + 1, 1)
    return carry

  lax.fori_loop(0, n_pairs, pair, 0)
  drain_o(0)
  drain_o(1)


_R = 2048


def _dinv_of(deg_ref):
  d = deg_ref[0, :] + deg_ref[1, :]
  return jnp.where(d > 0, lax.rsqrt(d), 0.0)


def _tc_prescale_body(deg_ref, x_ref, y_ref):
  dinv = _dinv_of(deg_ref)
  y_ref[...] = x_ref[...] * dinv[:, None]


def _tc_conv1_body(deg_ref, agg_ref, w_ref, b_ref, x1_ref, y2_ref):
  dinv = _dinv_of(deg_ref)
  agg = (agg_ref[0] + agg_ref[1]) * dinv[:, None]
  h = jnp.dot(agg, w_ref[...], preferred_element_type=jnp.float32) + b_ref[...]
  x1 = _leaky(h)
  x1_ref[...] = x1
  y2 = x1 * dinv[:, None]
  y2_ref[0] = y2[:, :16]
  y2_ref[1] = y2[:, 16:]


def _tc_conv2_body(deg_ref, agg_ref, x1_ref, w_ref, b_ref, x2_ref, y3_ref):
  dinv = _dinv_of(deg_ref)
  agg = jnp.concatenate([agg_ref[0], agg_ref[1]], axis=1) * dinv[:, None]
  h = jnp.dot(agg, w_ref[...], preferred_element_type=jnp.float32) + b_ref[...]
  x2 = _leaky(h) + x1_ref[...]
  x2_ref[...] = x2
  y3 = x2 * dinv[:, None]
  y3_ref[0] = y3[:, :16]
  y3_ref[1] = y3[:, 16:]


def _tc_conv3_body(deg_ref, agg_ref, w_ref, b_ref, x3_ref):
  dinv = _dinv_of(deg_ref)
  agg = jnp.concatenate([agg_ref[0], agg_ref[1]], axis=1) * dinv[:, None]
  x3_ref[...] = (
      jnp.dot(agg, w_ref[...], preferred_element_type=jnp.float32) + b_ref[...])


def _mlp3(h, w0, b0, w1, b1, w2, b2):
  h = _leaky(jnp.dot(h, w0, preferred_element_type=jnp.float32) + b0)
  h = _leaky(jnp.dot(h, w1, preferred_element_type=jnp.float32) + b1)
  return jnp.dot(h, w2, preferred_element_type=jnp.float32) + b2


def _tc_heads_body(x_ref, aw0, ab0, aw1, ab1, aw2, ab2,
                   bw0, bb0, bw1, bb1, bw2, bb2, oa_ref, ob_ref):
  h = x_ref[...]
  oa_ref[...] = _mlp3(h, aw0[...], ab0[...], aw1[...], ab1[...], aw2[...], ab2[...])
  ob_ref[...] = _mlp3(h, bw0[...], bb0[...], bw1[...], bb1[...], bw2[...], bb2[...])


def _full(shape):
  zeros = (0,) * len(shape)
  return pl.BlockSpec(shape, lambda i, z=zeros: z)


def _rows(shape, dim=0):
  def imap(i):
    idx = [0] * len(shape)
    idx[dim] = i
    return tuple(idx)
  return pl.BlockSpec(shape, imap)


def _run_heads(xarr, rows_total, rblock, p, n1, n2, d1, d2):
  grid = rows_total // rblock
  wspecs = []
  wargs = []
  for nm, dout in ((n1, d1), (n2, d2)):
    for li, (di, do) in enumerate(((64, 64), (64, 64), (64, dout))):
      wargs += [p[f'{nm}_w{li}'], p[f'{nm}_b{li}'].reshape(1, do)]
      wspecs += [_full((di, do)), _full((1, do))]
  return pl.pallas_call(
      _tc_heads_body,
      grid=(grid,),
      in_specs=[_rows((rblock, 64))] + wspecs,
      out_specs=[_rows((rblock, d1)), _rows((rblock, d2))],
      out_shape=[jax.ShapeDtypeStruct((rows_total, d1), jnp.float32),
                 jax.ShapeDtypeStruct((rows_total, d2), jnp.float32)],
  )(xarr, *wargs)


def kernel(x, edge_index, ud_edges, params):
  p = params
  f32 = jnp.float32

  dummy = lambda n: (jnp.arange(n, dtype=jnp.int32) % PAD_ROWS) + N
  src = jnp.concatenate([edge_index[0].astype(jnp.int32),
                         dummy(E_PAD - edge_index.shape[1])])
  dst = jnp.concatenate([edge_index[1].astype(jnp.int32),
                         dummy(E_PAD - edge_index.shape[1])])
  src2 = src.reshape(E_PAD // 128, 128)
  dst2 = dst.reshape(E_PAD // 128, 128)
  m = ud_edges.shape[0]
  ud0 = jnp.concatenate([ud_edges[:, 0].astype(jnp.int32), dummy(M_PAD - m)])
  ud1 = jnp.concatenate([ud_edges[:, 1].astype(jnp.int32), dummy(M_PAD - m)])
  ud0_2 = ud0.reshape(M_PAD // 128, 128)
  ud1_2 = ud1.reshape(M_PAD // 128, 128)
  x8 = jnp.pad(x[:, :6].astype(f32), ((0, N_PAD - N), (0, 2)))
  w0p = jnp.pad(p['gcn_w0'], ((0, 2), (0, 0)))
  z1 = jnp.zeros((ROWS_PER_SUB,), f32)
  z8 = jnp.zeros((ROWS_PER_SUB, 8), f32)
  z16 = jnp.zeros((ROWS_PER_SUB, 16), f32)

  deg2 = _sc_deg()(dst2, z1)

  y1 = pl.pallas_call(
      _tc_prescale_body,
      grid=(N_PAD // _R,),
      in_specs=[_rows((2, _R), dim=1), _rows((_R, 8))],
      out_specs=_rows((_R, 8)),
      out_shape=jax.ShapeDtypeStruct((N_PAD, 8), f32),
  )(deg2, x8)
  agg1 = _make_agg_edge_split(8)(src2, dst2, y1, z8)
  x1, y2 = pl.pallas_call(
      _tc_conv1_body,
      grid=(N_PAD // _R,),
      in_specs=[_rows((2, _R), dim=1), _rows((2, _R, 8), dim=1),
                _full((8, 32)), _full((1, 32))],
      out_specs=[_rows((_R, 32)), _rows((2, _R, 16), dim=1)],
      out_shape=[jax.ShapeDtypeStruct((N_PAD, 32), f32),
                 jax.ShapeDtypeStruct((2, N_PAD, 16), f32)],
  )(deg2, agg1, w0p, p['gcn_b0'].reshape(1, 32))

  agg2 = _sc_agg_feat_split()(src2, dst2, y2.reshape(2 * N_PAD, 16), z16)
  x2, y3 = pl.pallas_call(
      _tc_conv2_body,
      grid=(N_PAD // _R,),
      in_specs=[_rows((2, _R), dim=1), _rows((2, _R, 16), dim=1),
                _rows((_R, 32)), _full((32, 32)), _full((1, 32))],
      out_specs=[_rows((_R, 32)), _rows((2, _R, 16), dim=1)],
      out_shape=[jax.ShapeDtypeStruct((N_PAD, 32), f32),
                 jax.ShapeDtypeStruct((2, N_PAD, 16), f32)],
  )(deg2, agg2, x1, p['gcn_w1'], p['gcn_b1'].reshape(1, 32))

  agg3 = _sc_agg_feat_split()(src2, dst2, y3.reshape(2 * N_PAD, 16), z16)
  x3 = pl.pallas_call(
      _tc_conv3_body,
      grid=(N_PAD // _R,),
      in_specs=[_rows((2, _R), dim=1), _rows((2, _R, 16), dim=1),
                _full((32, 64)), _full((1, 64))],
      out_specs=_rows((_R, 64)),
      out_shape=jax.ShapeDtypeStruct((N_PAD, 64), f32),
  )(deg2, agg3, p['gcn_w2'], p['gcn_b2'].reshape(1, 64))

  ef = _sc_edge_feat()(ud0_2, ud1_2, x3)

  vp, vd = _run_heads(x3, N, 2000, p, 'pn', 'dn', 10, 10)
  ep, ed = _run_heads(ef, m, 2000, p, 'pe', 'de', 10, 16)

  return (x3[:N], vp, vd, ep, ed)


# speedup vs baseline: 15.1055x; 1.0146x over previous
import functools

import jax
import jax.numpy as jnp
from jax import lax
from jax.experimental import pallas as pl
from jax.experimental.pallas import tpu as pltpu
from jax.experimental.pallas import tpu_sc as plsc

N = 100000
N_PAD = 100352
PAD_ROWS = N_PAD - N
E_PAD = 1605632
M_PAD = 819200
NC, NS = 2, 16
K = 8
CHUNK = K * 128
ROWS_PER_SUB = N_PAD // NS
SLOPE = 0.1

@functools.lru_cache(maxsize=None)
def _mesh():
  return plsc.VectorSubcoreMesh(
      core_axis_name="c", subcore_axis_name="s", num_cores=NC, num_subcores=NS)


def _leaky(h):
  return jnp.where(h > 0, h, SLOPE * h)


KD = 4


@functools.lru_cache(maxsize=None)
def _sc_deg():
  return pl.kernel(
      _sc_deg_body,
      out_type=jax.ShapeDtypeStruct((NC, N_PAD), jnp.float32),
      mesh=_mesh(),
      compiler_params=pltpu.CompilerParams(use_tc_tiling_on_sc=False),
      scratch_types=[
          pltpu.VMEM((2, KD, 128), jnp.int32),
          pltpu.VMEM((128,), jnp.float32),
          pltpu.VMEM_SHARED((N_PAD,), jnp.float32),
          pltpu.SemaphoreType.DMA((2,)),
      ],
  )


def _sc_deg_body(dst2_hbm, zeros1_hbm, out_hbm, dst_v, ones_v, acc, ssem):
  c = lax.axis_index("c")
  s = lax.axis_index("s")
  wid = c * NS + s
  pltpu.sync_copy(zeros1_hbm, acc.at[pl.ds(s * ROWS_PER_SUB, ROWS_PER_SUB)])
  for i in range(8):
    ones_v[pl.ds(i * 16, 16)] = jnp.ones((16,), jnp.float32)
  plsc.subcore_barrier()
  worker_rows = E_PAD // 32 // 128
  row0 = wid * worker_rows
  n_pairs = worker_rows // KD // 2
  SB = KD * 128 * 4

  def load(ci, b):
    pltpu.sync_copy(dst2_hbm.at[pl.ds(row0 + ci * KD, KD)], dst_v.at[b])

  def issue_s(b):
    for j in range(KD):
      pltpu.async_copy(ones_v, acc.at[dst_v.at[b, j]], ssem.at[b], add=True)

  def drain_s(b):
    for j in range(KD):
      pltpu.make_async_copy(ones_v, acc.at[dst_v.at[b, j]], ssem.at[b]).wait()

  load(0, 0)

  def pair(p, carry):
    issue_s(0)
    @pl.when(p > 0)
    def _():
      drain_s(1)
    load(2 * p + 1, 1)
    issue_s(1)
    @pl.when(p < n_pairs - 1)
    def _():
      drain_s(0)
      load(2 * p + 2, 0)
    return carry

  lax.fori_loop(0, n_pairs, pair, 0)
  drain_s(0)
  drain_s(1)
  plsc.subcore_barrier()
  sl = pl.ds(s * ROWS_PER_SUB, ROWS_PER_SUB)
  pltpu.sync_copy(acc.at[sl], out_hbm.at[c, sl])


def _agg_body(src2_hbm, dst2_hbm, ytab_hbm, zeros_hbm, out_hbm,
              src_v, dst_v, rows_v, acc, gsem, ssem, *, width, k, feat_split):
  c = lax.axis_index("c")
  s = lax.axis_index("s")
  pltpu.sync_copy(zeros_hbm, acc.at[pl.ds(s * ROWS_PER_SUB, ROWS_PER_SUB)])
  plsc.subcore_barrier()
  if feat_split:
    worker_rows = E_PAD // NS // 128
    row0 = s * worker_rows
    bias = c * N_PAD
  else:
    worker_rows = E_PAD // 32 // 128
    row0 = (c * NS + s) * worker_rows
  n_pairs = worker_rows // k // 2
  GB = k * 128 * width * 4

  def load(ci, b):
    r = row0 + ci * k
    pltpu.sync_copy(src2_hbm.at[pl.ds(r, k)], src_v.at[b])
    pltpu.sync_copy(dst2_hbm.at[pl.ds(r, k)], dst_v.at[b])
    if feat_split:
      for j in range(k):
        for t in range(8):
          sl = (b, j, pl.ds(t * 16, 16))
          src_v[sl] = src_v[sl] + bias

  def issue_g(b):
    for j in range(k):
      pltpu.async_copy(ytab_hbm.at[src_v.at[b, j]],
                       rows_v.at[b, pl.ds(j * 128, 128)], gsem.at[b])

  def issue_s(b):
    for j in range(k):
      pltpu.async_copy(rows_v.at[b, pl.ds(j * 128, 128)],
                       acc.at[dst_v.at[b, j]], ssem.at[b], add=True)

  def drain_g(b):
    for j in range(k):
      pltpu.make_async_copy(ytab_hbm.at[src_v.at[b, j]],
                            rows_v.at[b, pl.ds(j * 128, 128)],
                            gsem.at[b]).wait()

  def drain_s(b):
    for j in range(k):
      pltpu.make_async_copy(rows_v.at[b, pl.ds(j * 128, 128)],
                            acc.at[dst_v.at[b, j]], ssem.at[b]).wait()

  load(0, 0)
  issue_g(0)

  def pair(p, carry):
    @pl.when(p > 0)
    def _():
      drain_s(1)
    load(2 * p + 1, 1)
    issue_g(1)
    drain_g(0)
    issue_s(0)
    @pl.when(p < n_pairs - 1)
    def _():
      drain_s(0)
      load(2 * p + 2, 0)
      issue_g(0)
    drain_g(1)
    issue_s(1)
    return carry

  lax.fori_loop(0, n_pairs, pair, 0)
  drain_s(0)
  drain_s(1)
  plsc.subcore_barrier()
  sl = pl.ds(s * ROWS_PER_SUB, ROWS_PER_SUB)
  pltpu.sync_copy(acc.at[sl], out_hbm.at[c, sl])


@functools.lru_cache(maxsize=None)
def _make_agg_edge_split(width):
  k = KD
  body = functools.partial(_agg_body, width=width, k=k, feat_split=False)
  return pl.kernel(
      body,
      out_type=jax.ShapeDtypeStruct((NC, N_PAD, width), jnp.float32),
      mesh=_mesh(),
      compiler_params=pltpu.CompilerParams(use_tc_tiling_on_sc=False),
      scratch_types=[
          pltpu.VMEM((2, k, 128), jnp.int32),
          pltpu.VMEM((2, k, 128), jnp.int32),
          pltpu.VMEM((2, k * 128, width), jnp.float32),
          pltpu.VMEM_SHARED((N_PAD, width), jnp.float32),
          pltpu.SemaphoreType.DMA((2,)),
          pltpu.SemaphoreType.DMA((2,)),
      ],
  )


@functools.lru_cache(maxsize=None)
def _sc_agg_feat_split():
  k = KD
  body = functools.partial(_agg_body, width=16, k=k, feat_split=True)
  return pl.kernel(
      body,
      out_type=jax.ShapeDtypeStruct((NC, N_PAD, 16), jnp.float32),
      mesh=_mesh(),
      compiler_params=pltpu.CompilerParams(use_tc_tiling_on_sc=False),
      scratch_types=[
          pltpu.VMEM((2, k, 128), jnp.int32),
          pltpu.VMEM((2, k, 128), jnp.int32),
          pltpu.VMEM((2, k * 128, 16), jnp.float32),
          pltpu.VMEM_SHARED((N_PAD, 16), jnp.float32),
          pltpu.SemaphoreType.DMA((2,)),
          pltpu.SemaphoreType.DMA((2,)),
      ],
  )


@functools.lru_cache(maxsize=None)
def _sc_edge_feat():
  return pl.kernel(
      _sc_edge_feat_body,
      out_type=jax.ShapeDtypeStruct((M_PAD, 64), jnp.float32),
      mesh=_mesh(),
      compiler_params=pltpu.CompilerParams(use_tc_tiling_on_sc=False),
      scratch_types=[
          pltpu.VMEM((2, KD, 128), jnp.int32),
          pltpu.VMEM((2, KD, 128), jnp.int32),
          pltpu.VMEM((2, KD * 128, 64), jnp.float32),
          pltpu.SemaphoreType.DMA((2,)),
          pltpu.SemaphoreType.DMA((2,)),
      ],
  )


def _sc_edge_feat_body(ud0_hbm, ud1_hbm, x_hbm, out_hbm,
                       i0_v, i1_v, rows_v, gsem, osem):
  c = lax.axis_index("c")
  s = lax.axis_index("s")
  wid = c * NS + s
  k = KD
  worker_rows = M_PAD // 32 // 128
  row0 = wid * worker_rows
  n_pairs = worker_rows // k // 2
  GB = k * 128 * 64 * 4

  def load(ci, b):
    r = row0 + ci * k
    pltpu.sync_copy(ud0_hbm.at[pl.ds(r, k)], i0_v.at[b])
    pltpu.sync_copy(ud1_hbm.at[pl.ds(r, k)], i1_v.at[b])

  def g1(b):
    for j in range(k):
      pltpu.async_copy(x_hbm.at[i0_v.at[b, j]],
                       rows_v.at[b, pl.ds(j * 128, 128)], gsem.at[b])

  def g2(b):
    for j in range(k):
      pltpu.async_copy(x_hbm.at[i1_v.at[b, j]],
                       rows_v.at[b, pl.ds(j * 128, 128)], gsem.at[b], add=True)

  def store(ci, b):
    pltpu.async_copy(rows_v.at[b],
                     out_hbm.at[pl.ds((row0 + ci * k) * 128, k * 128)],
                     osem.at[b])

  def drain_g(b, iv):
    for j in range(k):
      pltpu.make_async_copy(x_hbm.at[iv.at[b, j]],
                            rows_v.at[b, pl.ds(j * 128, 128)],
                            gsem.at[b]).wait()

  def drain_o(b):
    pltpu.make_async_copy(rows_v.at[b], out_hbm.at[pl.ds(0, k * 128)],
                          osem.at[b]).wait()

  load(0, 0)
  g1(0)

  def pair(p, carry):
    @pl.when(p > 0)
    def _():
      drain_o(1)
    load(2 * p + 1, 1)
    g1(1)
    drain_g(0, i0_v)
    g2(0)
    drain_g(0, i1_v)
    store(2 * p, 0)
    @pl.when(p < n_pairs - 1)
    def _():
      drain_o(0)
      load(2 * p + 2, 0)
      g1(0)
    drain_g(1, i0_v)
    g2(1)
    drain_g(1, i1_v)
    store(2 * p + 1, 1)
    return carry

  lax.fori_loop(0, n_pairs, pair, 0)
  drain_o(0)
  drain_o(1)


_R = 2048


def _dinv_of(deg_ref):
  d = deg_ref[0, :] + deg_ref[1, :]
  return jnp.where(d > 0, lax.rsqrt(d), 0.0)


def _tc_prescale_body(deg_ref, x_ref, y_ref):
  dinv = _dinv_of(deg_ref)
  y_ref[...] = x_ref[...] * dinv[:, None]


def _tc_conv1_body(deg_ref, agg_ref, w_ref, b_ref, x1_ref, y2_ref):
  dinv = _dinv_of(deg_ref)
  agg = (agg_ref[0] + agg_ref[1]) * dinv[:, None]
  h = jnp.dot(agg, w_ref[...], preferred_element_type=jnp.float32) + b_ref[...]
  x1 = _leaky(h)
  x1_ref[...] = x1
  y2 = x1 * dinv[:, None]
  y2_ref[0] = y2[:, :16]
  y2_ref[1] = y2[:, 16:]


def _tc_conv2_body(deg_ref, agg_ref, x1_ref, w_ref, b_ref, x2_ref, y3_ref):
  dinv = _dinv_of(deg_ref)
  agg = jnp.concatenate([agg_ref[0], agg_ref[1]], axis=1) * dinv[:, None]
  h = jnp.dot(agg, w_ref[...], preferred_element_type=jnp.float32) + b_ref[...]
  x2 = _leaky(h) + x1_ref[...]
  x2_ref[...] = x2
  y3 = x2 * dinv[:, None]
  y3_ref[0] = y3[:, :16]
  y3_ref[1] = y3[:, 16:]


def _tc_conv3_body(deg_ref, agg_ref, w_ref, b_ref, x3_ref):
  dinv = _dinv_of(deg_ref)
  agg = jnp.concatenate([agg_ref[0], agg_ref[1]], axis=1) * dinv[:, None]
  x3_ref[...] = (
      jnp.dot(agg, w_ref[...], preferred_element_type=jnp.float32) + b_ref[...])


def _mlp3(h, w0, b0, w1, b1, w2, b2):
  h = _leaky(jnp.dot(h, w0, preferred_element_type=jnp.float32) + b0)
  h = _leaky(jnp.dot(h, w1, preferred_element_type=jnp.float32) + b1)
  return jnp.dot(h, w2, preferred_element_type=jnp.float32) + b2


def _tc_heads_body(x_ref, w0, b0, w1, b1, w2, b2, oa_ref, ob_ref):
  d1 = oa_ref.shape[-1]
  o = _mlp3(x_ref[...], w0[...], b0[...], w1[...], b1[...], w2[...], b2[...])
  oa_ref[...] = o[:, :d1]
  ob_ref[...] = o[:, d1:]


def _full(shape):
  zeros = (0,) * len(shape)
  return pl.BlockSpec(shape, lambda i, z=zeros: z)


def _rows(shape, dim=0):
  def imap(i):
    idx = [0] * len(shape)
    idx[dim] = i
    return tuple(idx)
  return pl.BlockSpec(shape, imap)


def _blockdiag(a, b):
  za = jnp.zeros((a.shape[0], b.shape[1]), a.dtype)
  zb = jnp.zeros((b.shape[0], a.shape[1]), a.dtype)
  return jnp.concatenate([jnp.concatenate([a, za], 1),
                          jnp.concatenate([zb, b], 1)], 0)


def _run_heads(xarr, rows_total, rblock, p, n1, n2, d1, d2):
  grid = rows_total // rblock
  w0 = jnp.concatenate([p[f'{n1}_w0'], p[f'{n2}_w0']], axis=1)
  b0 = jnp.concatenate([p[f'{n1}_b0'], p[f'{n2}_b0']]).reshape(1, 128)
  w1 = _blockdiag(p[f'{n1}_w1'], p[f'{n2}_w1'])
  b1 = jnp.concatenate([p[f'{n1}_b1'], p[f'{n2}_b1']]).reshape(1, 128)
  w2 = _blockdiag(p[f'{n1}_w2'], p[f'{n2}_w2'])
  b2 = jnp.concatenate([p[f'{n1}_b2'], p[f'{n2}_b2']]).reshape(1, d1 + d2)
  wargs = [w0, b0, w1, b1, w2, b2]
  wspecs = [_full(w.shape) for w in wargs]
  return pl.pallas_call(
      _tc_heads_body,
      grid=(grid,),
      in_specs=[_rows((rblock, 64))] + wspecs,
      out_specs=[_rows((rblock, d1)), _rows((rblock, d2))],
      out_shape=[jax.ShapeDtypeStruct((rows_total, d1), jnp.float32),
                 jax.ShapeDtypeStruct((rows_total, d2), jnp.float32)],
  )(xarr, *wargs)


def kernel(x, edge_index, ud_edges, params):
  p = params
  f32 = jnp.float32

  dummy = lambda n: (jnp.arange(n, dtype=jnp.int32) % PAD_ROWS) + N
  src = jnp.concatenate([edge_index[0].astype(jnp.int32),
                         dummy(E_PAD - edge_index.shape[1])])
  dst = jnp.concatenate([edge_index[1].astype(jnp.int32),
                         dummy(E_PAD - edge_index.shape[1])])
  src2 = src.reshape(E_PAD // 128, 128)
  dst2 = dst.reshape(E_PAD // 128, 128)
  m = ud_edges.shape[0]
  ud0 = jnp.concatenate([ud_edges[:, 0].astype(jnp.int32), dummy(M_PAD - m)])
  ud1 = jnp.concatenate([ud_edges[:, 1].astype(jnp.int32), dummy(M_PAD - m)])
  ud0_2 = ud0.reshape(M_PAD // 128, 128)
  ud1_2 = ud1.reshape(M_PAD // 128, 128)
  x8 = jnp.pad(x[:, :6].astype(f32), ((0, N_PAD - N), (0, 2)))
  w0p = jnp.pad(p['gcn_w0'], ((0, 2), (0, 0)))
  z1 = jnp.zeros((ROWS_PER_SUB,), f32)
  z8 = jnp.zeros((ROWS_PER_SUB, 8), f32)
  z16 = jnp.zeros((ROWS_PER_SUB, 16), f32)

  deg2 = _sc_deg()(dst2, z1)

  y1 = pl.pallas_call(
      _tc_prescale_body,
      grid=(N_PAD // _R,),
      in_specs=[_rows((2, _R), dim=1), _rows((_R, 8))],
      out_specs=_rows((_R, 8)),
      out_shape=jax.ShapeDtypeStruct((N_PAD, 8), f32),
  )(deg2, x8)
  agg1 = _make_agg_edge_split(8)(src2, dst2, y1, z8)
  x1, y2 = pl.pallas_call(
      _tc_conv1_body,
      grid=(N_PAD // _R,),
      in_specs=[_rows((2, _R), dim=1), _rows((2, _R, 8), dim=1),
                _full((8, 32)), _full((1, 32))],
      out_specs=[_rows((_R, 32)), _rows((2, _R, 16), dim=1)],
      out_shape=[jax.ShapeDtypeStruct((N_PAD, 32), f32),
                 jax.ShapeDtypeStruct((2, N_PAD, 16), f32)],
  )(deg2, agg1, w0p, p['gcn_b0'].reshape(1, 32))

  agg2 = _sc_agg_feat_split()(src2, dst2, y2.reshape(2 * N_PAD, 16), z16)
  x2, y3 = pl.pallas_call(
      _tc_conv2_body,
      grid=(N_PAD // _R,),
      in_specs=[_rows((2, _R), dim=1), _rows((2, _R, 16), dim=1),
                _rows((_R, 32)), _full((32, 32)), _full((1, 32))],
      out_specs=[_rows((_R, 32)), _rows((2, _R, 16), dim=1)],
      out_shape=[jax.ShapeDtypeStruct((N_PAD, 32), f32),
                 jax.ShapeDtypeStruct((2, N_PAD, 16), f32)],
  )(deg2, agg2, x1, p['gcn_w1'], p['gcn_b1'].reshape(1, 32))

  agg3 = _sc_agg_feat_split()(src2, dst2, y3.reshape(2 * N_PAD, 16), z16)
  x3 = pl.pallas_call(
      _tc_conv3_body,
      grid=(N_PAD // _R,),
      in_specs=[_rows((2, _R), dim=1), _rows((2, _R, 16), dim=1),
                _full((32, 64)), _full((1, 64))],
      out_specs=_rows((_R, 64)),
      out_shape=jax.ShapeDtypeStruct((N_PAD, 64), f32),
  )(deg2, agg3, p['gcn_w2'], p['gcn_b2'].reshape(1, 64))

  ef = _sc_edge_feat()(ud0_2, ud1_2, x3)

  vp, vd = _run_heads(x3, N, 2000, p, 'pn', 'dn', 10, 10)
  ep, ed = _run_heads(ef, m, 2000, p, 'pe', 'de', 10, 16)

  return (x3[:N], vp, vd, ep, ed)


# speedup vs baseline: 16.3158x; 1.0801x over previous
import functools

import jax
import jax.numpy as jnp
from jax import lax
from jax.experimental import pallas as pl
from jax.experimental.pallas import tpu as pltpu
from jax.experimental.pallas import tpu_sc as plsc

N = 100000
N_PAD = 100352
PAD_ROWS = N_PAD - N
E_PAD = 1605632
M_PAD = 819200
NC, NS = 2, 16
K = 8
CHUNK = K * 128
ROWS_PER_SUB = N_PAD // NS
SLOPE = 0.1

@functools.lru_cache(maxsize=None)
def _mesh():
  return plsc.VectorSubcoreMesh(
      core_axis_name="c", subcore_axis_name="s", num_cores=NC, num_subcores=NS)


def _leaky(h):
  return jnp.where(h > 0, h, SLOPE * h)


KD = 4


@functools.lru_cache(maxsize=None)
def _sc_deg():
  return pl.kernel(
      _sc_deg_body,
      out_type=jax.ShapeDtypeStruct((NC, N_PAD), jnp.float32),
      mesh=_mesh(),
      compiler_params=pltpu.CompilerParams(use_tc_tiling_on_sc=False),
      scratch_types=[
          pltpu.VMEM((2, KD, 128), jnp.int32),
          pltpu.VMEM((128,), jnp.float32),
          pltpu.VMEM_SHARED((N_PAD,), jnp.float32),
          pltpu.SemaphoreType.DMA((2,)),
      ],
  )


def _sc_deg_body(dst2_hbm, zeros1_hbm, out_hbm, dst_v, ones_v, acc, ssem):
  c = lax.axis_index("c")
  s = lax.axis_index("s")
  wid = c * NS + s
  pltpu.sync_copy(zeros1_hbm, acc.at[pl.ds(s * ROWS_PER_SUB, ROWS_PER_SUB)])
  for i in range(8):
    ones_v[pl.ds(i * 16, 16)] = jnp.ones((16,), jnp.float32)
  plsc.subcore_barrier()
  worker_rows = E_PAD // 32 // 128
  row0 = wid * worker_rows
  n_pairs = worker_rows // KD // 2
  SB = KD * 128 * 4

  def load(ci, b):
    pltpu.sync_copy(dst2_hbm.at[pl.ds(row0 + ci * KD, KD)], dst_v.at[b])

  def issue_s(b):
    for j in range(KD):
      pltpu.async_copy(ones_v, acc.at[dst_v.at[b, j]], ssem.at[b], add=True)

  def drain_s(b):
    for j in range(KD):
      pltpu.make_async_copy(ones_v, acc.at[dst_v.at[b, j]], ssem.at[b]).wait()

  load(0, 0)

  def pair(p, carry):
    issue_s(0)
    @pl.when(p > 0)
    def _():
      drain_s(1)
    load(2 * p + 1, 1)
    issue_s(1)
    @pl.when(p < n_pairs - 1)
    def _():
      drain_s(0)
      load(2 * p + 2, 0)
    return carry

  lax.fori_loop(0, n_pairs, pair, 0)
  drain_s(0)
  drain_s(1)
  plsc.subcore_barrier()
  sl = pl.ds(s * ROWS_PER_SUB, ROWS_PER_SUB)
  pltpu.sync_copy(acc.at[sl], out_hbm.at[c, sl])


def _agg_body(src2_hbm, dst2_hbm, ytab_hbm, zeros_hbm, out_hbm,
              src_v, dst_v, rows_v, acc, gsem, ssem, *, width, k, feat_split):
  c = lax.axis_index("c")
  s = lax.axis_index("s")
  pltpu.sync_copy(zeros_hbm, acc.at[pl.ds(s * ROWS_PER_SUB, ROWS_PER_SUB)])
  plsc.subcore_barrier()
  if feat_split:
    worker_rows = E_PAD // NS // 128
    row0 = s * worker_rows
    bias = c * N_PAD
  else:
    worker_rows = E_PAD // 32 // 128
    row0 = (c * NS + s) * worker_rows
  n_pairs = worker_rows // k // 2
  GB = k * 128 * width * 4

  def load(ci, b):
    r = row0 + ci * k
    pltpu.sync_copy(src2_hbm.at[pl.ds(r, k)], src_v.at[b])
    pltpu.sync_copy(dst2_hbm.at[pl.ds(r, k)], dst_v.at[b])
    if feat_split:
      for j in range(k):
        for t in range(8):
          sl = (b, j, pl.ds(t * 16, 16))
          src_v[sl] = src_v[sl] + bias

  def issue_g(b):
    for j in range(k):
      pltpu.async_copy(ytab_hbm.at[src_v.at[b, j]],
                       rows_v.at[b, pl.ds(j * 128, 128)], gsem.at[b])

  def issue_s(b):
    for j in range(k):
      pltpu.async_copy(rows_v.at[b, pl.ds(j * 128, 128)],
                       acc.at[dst_v.at[b, j]], ssem.at[b], add=True)

  def drain_g(b):
    for j in range(k):
      pltpu.make_async_copy(ytab_hbm.at[src_v.at[b, j]],
                            rows_v.at[b, pl.ds(j * 128, 128)],
                            gsem.at[b]).wait()

  def drain_s(b):
    for j in range(k):
      pltpu.make_async_copy(rows_v.at[b, pl.ds(j * 128, 128)],
                            acc.at[dst_v.at[b, j]], ssem.at[b]).wait()

  load(0, 0)
  issue_g(0)

  def pair(p, carry):
    @pl.when(p > 0)
    def _():
      drain_s(1)
    load(2 * p + 1, 1)
    issue_g(1)
    drain_g(0)
    issue_s(0)
    @pl.when(p < n_pairs - 1)
    def _():
      drain_s(0)
      load(2 * p + 2, 0)
      issue_g(0)
    drain_g(1)
    issue_s(1)
    return carry

  lax.fori_loop(0, n_pairs, pair, 0)
  drain_s(0)
  drain_s(1)
  plsc.subcore_barrier()
  sl = pl.ds(s * ROWS_PER_SUB, ROWS_PER_SUB)
  pltpu.sync_copy(acc.at[sl], out_hbm.at[c, sl])


@functools.lru_cache(maxsize=None)
def _make_agg_edge_split(width):
  k = KD
  body = functools.partial(_agg_body, width=width, k=k, feat_split=False)
  return pl.kernel(
      body,
      out_type=jax.ShapeDtypeStruct((NC, N_PAD, width), jnp.float32),
      mesh=_mesh(),
      compiler_params=pltpu.CompilerParams(use_tc_tiling_on_sc=False),
      scratch_types=[
          pltpu.VMEM((2, k, 128), jnp.int32),
          pltpu.VMEM((2, k, 128), jnp.int32),
          pltpu.VMEM((2, k * 128, width), jnp.float32),
          pltpu.VMEM_SHARED((N_PAD, width), jnp.float32),
          pltpu.SemaphoreType.DMA((2,)),
          pltpu.SemaphoreType.DMA((2,)),
      ],
  )


@functools.lru_cache(maxsize=None)
def _sc_agg_feat_split():
  k = KD
  body = functools.partial(_agg_body, width=16, k=k, feat_split=True)
  return pl.kernel(
      body,
      out_type=jax.ShapeDtypeStruct((NC, N_PAD, 16), jnp.float32),
      mesh=_mesh(),
      compiler_params=pltpu.CompilerParams(use_tc_tiling_on_sc=False),
      scratch_types=[
          pltpu.VMEM((2, k, 128), jnp.int32),
          pltpu.VMEM((2, k, 128), jnp.int32),
          pltpu.VMEM((2, k * 128, 16), jnp.float32),
          pltpu.VMEM_SHARED((N_PAD, 16), jnp.float32),
          pltpu.SemaphoreType.DMA((2,)),
          pltpu.SemaphoreType.DMA((2,)),
      ],
  )


@functools.lru_cache(maxsize=None)
def _sc_edge_feat():
  return pl.kernel(
      _sc_edge_feat_body,
      out_type=jax.ShapeDtypeStruct((M_PAD, 64), jnp.float32),
      mesh=_mesh(),
      compiler_params=pltpu.CompilerParams(use_tc_tiling_on_sc=False),
      scratch_types=[
          pltpu.VMEM((2, KD, 128), jnp.int32),
          pltpu.VMEM((2, KD, 128), jnp.int32),
          pltpu.VMEM((2, KD * 128, 64), jnp.float32),
          pltpu.SemaphoreType.DMA((2,)),
          pltpu.SemaphoreType.DMA((2,)),
      ],
  )


def _sc_edge_feat_body(ud0_hbm, ud1_hbm, x_hbm, out_hbm,
                       i0_v, i1_v, rows_v, gsem, osem):
  c = lax.axis_index("c")
  s = lax.axis_index("s")
  wid = c * NS + s
  k = KD
  worker_rows = M_PAD // 32 // 128
  row0 = wid * worker_rows
  n_pairs = worker_rows // k // 2
  GB = k * 128 * 64 * 4

  def load(ci, b):
    r = row0 + ci * k
    pltpu.sync_copy(ud0_hbm.at[pl.ds(r, k)], i0_v.at[b])
    pltpu.sync_copy(ud1_hbm.at[pl.ds(r, k)], i1_v.at[b])

  def g1(b):
    for j in range(k):
      pltpu.async_copy(x_hbm.at[i0_v.at[b, j]],
                       rows_v.at[b, pl.ds(j * 128, 128)], gsem.at[b])

  def g2(b):
    for j in range(k):
      pltpu.async_copy(x_hbm.at[i1_v.at[b, j]],
                       rows_v.at[b, pl.ds(j * 128, 128)], gsem.at[b], add=True)

  def store(ci, b):
    pltpu.async_copy(rows_v.at[b],
                     out_hbm.at[pl.ds((row0 + ci * k) * 128, k * 128)],
                     osem.at[b])

  def drain_g(b, iv):
    for j in range(k):
      pltpu.make_async_copy(x_hbm.at[iv.at[b, j]],
                            rows_v.at[b, pl.ds(j * 128, 128)],
                            gsem.at[b]).wait()

  def drain_o(b):
    pltpu.make_async_copy(rows_v.at[b], out_hbm.at[pl.ds(0, k * 128)],
                          osem.at[b]).wait()

  load(0, 0)
  g1(0)

  def pair(p, carry):
    @pl.when(p > 0)
    def _():
      drain_o(1)
    load(2 * p + 1, 1)
    g1(1)
    drain_g(0, i0_v)
    g2(0)
    drain_g(0, i1_v)
    store(2 * p, 0)
    @pl.when(p < n_pairs - 1)
    def _():
      drain_o(0)
      load(2 * p + 2, 0)
      g1(0)
    drain_g(1, i0_v)
    g2(1)
    drain_g(1, i1_v)
    store(2 * p + 1, 1)
    return carry

  lax.fori_loop(0, n_pairs, pair, 0)
  drain_o(0)
  drain_o(1)


_R = 2048


def _dinv_of(deg_ref):
  d = deg_ref[0, :] + deg_ref[1, :]
  return jnp.where(d > 0, lax.rsqrt(d), 0.0)


def _tc_prescale_body(deg_ref, x_ref, y_ref):
  dinv = _dinv_of(deg_ref)
  y_ref[...] = x_ref[...] * dinv[:, None]


def _tc_conv1_body(deg_ref, agg_ref, w_ref, b_ref, x1_ref, y2_ref):
  dinv = _dinv_of(deg_ref)
  agg = (agg_ref[0] + agg_ref[1]) * dinv[:, None]
  h = jnp.dot(agg, w_ref[...], preferred_element_type=jnp.float32) + b_ref[...]
  x1 = _leaky(h)
  x1_ref[...] = x1
  y2 = x1 * dinv[:, None]
  y2_ref[0] = y2[:, :16]
  y2_ref[1] = y2[:, 16:]


def _tc_conv2_body(deg_ref, agg_ref, x1_ref, w_ref, b_ref, x2_ref, y3_ref):
  dinv = _dinv_of(deg_ref)
  agg = jnp.concatenate([agg_ref[0], agg_ref[1]], axis=1) * dinv[:, None]
  h = jnp.dot(agg, w_ref[...], preferred_element_type=jnp.float32) + b_ref[...]
  x2 = _leaky(h) + x1_ref[...]
  x2_ref[...] = x2
  y3 = x2 * dinv[:, None]
  y3_ref[0] = y3[:, :16]
  y3_ref[1] = y3[:, 16:]


def _tc_conv3_body(deg_ref, agg_ref, w_ref, b_ref, x3_ref):
  dinv = _dinv_of(deg_ref)
  agg = jnp.concatenate([agg_ref[0], agg_ref[1]], axis=1) * dinv[:, None]
  x3_ref[...] = (
      jnp.dot(agg, w_ref[...], preferred_element_type=jnp.float32) + b_ref[...])


def _mlp3(h, w0, b0, w1, b1, w2, b2):
  h = _leaky(jnp.dot(h, w0, preferred_element_type=jnp.float32) + b0)
  h = _leaky(jnp.dot(h, w1, preferred_element_type=jnp.float32) + b1)
  return jnp.dot(h, w2, preferred_element_type=jnp.float32) + b2


def _tc_heads_body(x_ref, w0, b0, w1, b1, w2, b2, o_ref):
  o_ref[...] = _mlp3(x_ref[...], w0[...], b0[...], w1[...], b1[...],
                     w2[...], b2[...])


def _full(shape):
  zeros = (0,) * len(shape)
  return pl.BlockSpec(shape, lambda i, z=zeros: z)


def _rows(shape, dim=0):
  def imap(i):
    idx = [0] * len(shape)
    idx[dim] = i
    return tuple(idx)
  return pl.BlockSpec(shape, imap)


def _blockdiag(a, b):
  za = jnp.zeros((a.shape[0], b.shape[1]), a.dtype)
  zb = jnp.zeros((b.shape[0], a.shape[1]), a.dtype)
  return jnp.concatenate([jnp.concatenate([a, za], 1),
                          jnp.concatenate([zb, b], 1)], 0)


def _run_heads(xpacked, pairs_total, rblock, p, n1, n2, d1, d2):
  dsum = d1 + d2
  w0c = jnp.concatenate([p[f'{n1}_w0'], p[f'{n2}_w0']], axis=1)
  b0c = jnp.concatenate([p[f'{n1}_b0'], p[f'{n2}_b0']])
  w1c = _blockdiag(p[f'{n1}_w1'], p[f'{n2}_w1'])
  b1c = jnp.concatenate([p[f'{n1}_b1'], p[f'{n2}_b1']])
  w2c = _blockdiag(p[f'{n1}_w2'], p[f'{n2}_w2'])
  b2c = jnp.concatenate([p[f'{n1}_b2'], p[f'{n2}_b2']])
  w0 = _blockdiag(w0c, w0c)
  b0 = jnp.tile(b0c, 2).reshape(1, 256)
  w1 = _blockdiag(w1c, w1c)
  b1 = jnp.tile(b1c, 2).reshape(1, 256)
  w2 = jnp.zeros((256, 128), jnp.float32)
  w2 = w2.at[:128, :dsum].set(w2c).at[128:, 64:64 + dsum].set(w2c)
  b2 = jnp.zeros((128,), jnp.float32)
  b2 = b2.at[:dsum].set(b2c).at[64:64 + dsum].set(b2c).reshape(1, 128)
  wargs = [w0, b0, w1, b1, w2, b2]
  wspecs = [_full(w.shape) for w in wargs]
  out = pl.pallas_call(
      _tc_heads_body,
      grid=(pairs_total // rblock,),
      in_specs=[_rows((rblock, 128))] + wspecs,
      out_specs=_rows((rblock, 128)),
      out_shape=jax.ShapeDtypeStruct((pairs_total, 128), jnp.float32),
  )(xpacked, *wargs)
  t = out.reshape(pairs_total, 2, 64)
  oa = t[:, :, :d1].reshape(2 * pairs_total, d1)
  ob = t[:, :, d1:d1 + d2].reshape(2 * pairs_total, d2)
  return oa, ob


def kernel(x, edge_index, ud_edges, params):
  p = params
  f32 = jnp.float32

  dummy = lambda n: (jnp.arange(n, dtype=jnp.int32) % PAD_ROWS) + N
  src = jnp.concatenate([edge_index[0].astype(jnp.int32),
                         dummy(E_PAD - edge_index.shape[1])])
  dst = jnp.concatenate([edge_index[1].astype(jnp.int32),
                         dummy(E_PAD - edge_index.shape[1])])
  src2 = src.reshape(E_PAD // 128, 128)
  dst2 = dst.reshape(E_PAD // 128, 128)
  m = ud_edges.shape[0]
  ud0 = jnp.concatenate([ud_edges[:, 0].astype(jnp.int32), dummy(M_PAD - m)])
  ud1 = jnp.concatenate([ud_edges[:, 1].astype(jnp.int32), dummy(M_PAD - m)])
  ud0_2 = ud0.reshape(M_PAD // 128, 128)
  ud1_2 = ud1.reshape(M_PAD // 128, 128)
  x8 = jnp.pad(x[:, :6].astype(f32), ((0, N_PAD - N), (0, 2)))
  w0p = jnp.pad(p['gcn_w0'], ((0, 2), (0, 0)))
  z1 = jnp.zeros((ROWS_PER_SUB,), f32)
  z8 = jnp.zeros((ROWS_PER_SUB, 8), f32)
  z16 = jnp.zeros((ROWS_PER_SUB, 16), f32)

  deg2 = _sc_deg()(dst2, z1)

  y1 = pl.pallas_call(
      _tc_prescale_body,
      grid=(N_PAD // _R,),
      in_specs=[_rows((2, _R), dim=1), _rows((_R, 8))],
      out_specs=_rows((_R, 8)),
      out_shape=jax.ShapeDtypeStruct((N_PAD, 8), f32),
  )(deg2, x8)
  agg1 = _make_agg_edge_split(8)(src2, dst2, y1, z8)
  x1, y2 = pl.pallas_call(
      _tc_conv1_body,
      grid=(N_PAD // _R,),
      in_specs=[_rows((2, _R), dim=1), _rows((2, _R, 8), dim=1),
                _full((8, 32)), _full((1, 32))],
      out_specs=[_rows((_R, 32)), _rows((2, _R, 16), dim=1)],
      out_shape=[jax.ShapeDtypeStruct((N_PAD, 32), f32),
                 jax.ShapeDtypeStruct((2, N_PAD, 16), f32)],
  )(deg2, agg1, w0p, p['gcn_b0'].reshape(1, 32))

  agg2 = _sc_agg_feat_split()(src2, dst2, y2.reshape(2 * N_PAD, 16), z16)
  x2, y3 = pl.pallas_call(
      _tc_conv2_body,
      grid=(N_PAD // _R,),
      in_specs=[_rows((2, _R), dim=1), _rows((2, _R, 16), dim=1),
                _rows((_R, 32)), _full((32, 32)), _full((1, 32))],
      out_specs=[_rows((_R, 32)), _rows((2, _R, 16), dim=1)],
      out_shape=[jax.ShapeDtypeStruct((N_PAD, 32), f32),
                 jax.ShapeDtypeStruct((2, N_PAD, 16), f32)],
  )(deg2, agg2, x1, p['gcn_w1'], p['gcn_b1'].reshape(1, 32))

  agg3 = _sc_agg_feat_split()(src2, dst2, y3.reshape(2 * N_PAD, 16), z16)
  x3 = pl.pallas_call(
      _tc_conv3_body,
      grid=(N_PAD // _R,),
      in_specs=[_rows((2, _R), dim=1), _rows((2, _R, 16), dim=1),
                _full((32, 64)), _full((1, 64))],
      out_specs=_rows((_R, 64)),
      out_shape=jax.ShapeDtypeStruct((N_PAD, 64), f32),
  )(deg2, agg3, p['gcn_w2'], p['gcn_b2'].reshape(1, 64))

  ef = _sc_edge_feat()(ud0_2, ud1_2, x3)

  vp, vd = _run_heads(x3.reshape(N_PAD // 2, 128), N // 2, 1000,
                      p, 'pn', 'dn', 10, 10)
  ep, ed = _run_heads(ef.reshape(M_PAD // 2, 128), m // 2, 1000,
                      p, 'pe', 'de', 10, 16)

  return (x3[:N], vp, vd, ep, ed)


# speedup vs baseline: 17.0597x; 1.0456x over previous
import functools

import jax
import jax.numpy as jnp
from jax import lax
from jax.experimental import pallas as pl
from jax.experimental.pallas import tpu as pltpu
from jax.experimental.pallas import tpu_sc as plsc

N = 100000
N_PAD = 100352
PAD_ROWS = N_PAD - N
E_PAD = 1605632
M_PAD = 819200
NC, NS = 2, 16
K = 8
CHUNK = K * 128
ROWS_PER_SUB = N_PAD // NS
SLOPE = 0.1

@functools.lru_cache(maxsize=None)
def _mesh():
  return plsc.VectorSubcoreMesh(
      core_axis_name="c", subcore_axis_name="s", num_cores=NC, num_subcores=NS)


def _leaky(h):
  return jnp.where(h > 0, h, SLOPE * h)


KD = 4


@functools.lru_cache(maxsize=None)
def _sc_deg():
  return pl.kernel(
      _sc_deg_body,
      out_type=jax.ShapeDtypeStruct((NC, N_PAD), jnp.float32),
      mesh=_mesh(),
      compiler_params=pltpu.CompilerParams(use_tc_tiling_on_sc=False),
      scratch_types=[
          pltpu.VMEM((2, KD, 128), jnp.int32),
          pltpu.VMEM((128,), jnp.float32),
          pltpu.VMEM_SHARED((N_PAD,), jnp.float32),
          pltpu.SemaphoreType.DMA((2,)),
      ],
  )


def _sc_deg_body(dst2_hbm, zeros1_hbm, out_hbm, dst_v, ones_v, acc, ssem):
  c = lax.axis_index("c")
  s = lax.axis_index("s")
  wid = c * NS + s
  pltpu.sync_copy(zeros1_hbm, acc.at[pl.ds(s * ROWS_PER_SUB, ROWS_PER_SUB)])
  for i in range(8):
    ones_v[pl.ds(i * 16, 16)] = jnp.ones((16,), jnp.float32)
  plsc.subcore_barrier()
  worker_rows = E_PAD // 32 // 128
  row0 = wid * worker_rows
  n_pairs = worker_rows // KD // 2
  SB = KD * 128 * 4

  def load(ci, b):
    pltpu.sync_copy(dst2_hbm.at[pl.ds(row0 + ci * KD, KD)], dst_v.at[b])

  def issue_s(b):
    for j in range(KD):
      pltpu.async_copy(ones_v, acc.at[dst_v.at[b, j]], ssem.at[b], add=True)

  def drain_s(b):
    for j in range(KD):
      pltpu.make_async_copy(ones_v, acc.at[dst_v.at[b, j]], ssem.at[b]).wait()

  load(0, 0)

  def pair(p, carry):
    issue_s(0)
    @pl.when(p > 0)
    def _():
      drain_s(1)
    load(2 * p + 1, 1)
    issue_s(1)
    @pl.when(p < n_pairs - 1)
    def _():
      drain_s(0)
      load(2 * p + 2, 0)
    return carry

  lax.fori_loop(0, n_pairs, pair, 0)
  drain_s(0)
  drain_s(1)
  plsc.subcore_barrier()
  sl = pl.ds(s * ROWS_PER_SUB, ROWS_PER_SUB)
  pltpu.sync_copy(acc.at[sl], out_hbm.at[c, sl])


def _agg_body(src2_hbm, dst2_hbm, ytab_hbm, zeros_hbm, out_hbm,
              src_v, dst_v, rows_v, acc, gsem, ssem, *, width, k, feat_split):
  c = lax.axis_index("c")
  s = lax.axis_index("s")
  pltpu.sync_copy(zeros_hbm, acc.at[pl.ds(s * ROWS_PER_SUB, ROWS_PER_SUB)])
  plsc.subcore_barrier()
  if feat_split:
    worker_rows = E_PAD // NS // 128
    row0 = s * worker_rows
    bias = c * N_PAD
  else:
    worker_rows = E_PAD // 32 // 128
    row0 = (c * NS + s) * worker_rows
  n_pairs = worker_rows // k // 2
  GB = k * 128 * width * 4

  def load(ci, b):
    r = row0 + ci * k
    pltpu.sync_copy(src2_hbm.at[pl.ds(r, k)], src_v.at[b])
    pltpu.sync_copy(dst2_hbm.at[pl.ds(r, k)], dst_v.at[b])
    if feat_split:
      for j in range(k):
        for t in range(8):
          sl = (b, j, pl.ds(t * 16, 16))
          src_v[sl] = src_v[sl] + bias

  def issue_g(b):
    for j in range(k):
      pltpu.async_copy(ytab_hbm.at[src_v.at[b, j]],
                       rows_v.at[b, pl.ds(j * 128, 128)], gsem.at[b])

  def issue_s(b):
    for j in range(k):
      pltpu.async_copy(rows_v.at[b, pl.ds(j * 128, 128)],
                       acc.at[dst_v.at[b, j]], ssem.at[b], add=True)

  def drain_g(b):
    for j in range(k):
      pltpu.make_async_copy(ytab_hbm.at[src_v.at[b, j]],
                            rows_v.at[b, pl.ds(j * 128, 128)],
                            gsem.at[b]).wait()

  def drain_s(b):
    for j in range(k):
      pltpu.make_async_copy(rows_v.at[b, pl.ds(j * 128, 128)],
                            acc.at[dst_v.at[b, j]], ssem.at[b]).wait()

  load(0, 0)
  issue_g(0)

  def pair(p, carry):
    @pl.when(p > 0)
    def _():
      drain_s(1)
    load(2 * p + 1, 1)
    issue_g(1)
    drain_g(0)
    issue_s(0)
    @pl.when(p < n_pairs - 1)
    def _():
      drain_s(0)
      load(2 * p + 2, 0)
      issue_g(0)
    drain_g(1)
    issue_s(1)
    return carry

  lax.fori_loop(0, n_pairs, pair, 0)
  drain_s(0)
  drain_s(1)
  plsc.subcore_barrier()
  sl = pl.ds(s * ROWS_PER_SUB, ROWS_PER_SUB)
  pltpu.sync_copy(acc.at[sl], out_hbm.at[sl, pl.ds(c * width, width)])


@functools.lru_cache(maxsize=None)
def _make_agg_edge_split(width):
  k = KD
  body = functools.partial(_agg_body, width=width, k=k, feat_split=False)
  return pl.kernel(
      body,
      out_type=jax.ShapeDtypeStruct((N_PAD, 128), jnp.float32),
      mesh=_mesh(),
      compiler_params=pltpu.CompilerParams(use_tc_tiling_on_sc=False),
      scratch_types=[
          pltpu.VMEM((2, k, 128), jnp.int32),
          pltpu.VMEM((2, k, 128), jnp.int32),
          pltpu.VMEM((2, k * 128, width), jnp.float32),
          pltpu.VMEM_SHARED((N_PAD, width), jnp.float32),
          pltpu.SemaphoreType.DMA((2,)),
          pltpu.SemaphoreType.DMA((2,)),
      ],
  )


@functools.lru_cache(maxsize=None)
def _sc_agg_feat_split():
  k = KD
  body = functools.partial(_agg_body, width=16, k=k, feat_split=True)
  return pl.kernel(
      body,
      out_type=jax.ShapeDtypeStruct((N_PAD, 128), jnp.float32),
      mesh=_mesh(),
      compiler_params=pltpu.CompilerParams(use_tc_tiling_on_sc=False),
      scratch_types=[
          pltpu.VMEM((2, k, 128), jnp.int32),
          pltpu.VMEM((2, k, 128), jnp.int32),
          pltpu.VMEM((2, k * 128, 16), jnp.float32),
          pltpu.VMEM_SHARED((N_PAD, 16), jnp.float32),
          pltpu.SemaphoreType.DMA((2,)),
          pltpu.SemaphoreType.DMA((2,)),
      ],
  )


@functools.lru_cache(maxsize=None)
def _sc_edge_feat():
  return pl.kernel(
      _sc_edge_feat_body,
      out_type=jax.ShapeDtypeStruct((M_PAD, 64), jnp.float32),
      mesh=_mesh(),
      compiler_params=pltpu.CompilerParams(use_tc_tiling_on_sc=False),
      scratch_types=[
          pltpu.VMEM((2, KD, 128), jnp.int32),
          pltpu.VMEM((2, KD, 128), jnp.int32),
          pltpu.VMEM((2, KD * 128, 64), jnp.float32),
          pltpu.SemaphoreType.DMA((2, KD)),
          pltpu.SemaphoreType.DMA((2,)),
      ],
  )


def _sc_edge_feat_body(ud0_hbm, ud1_hbm, x_hbm, out_hbm,
                       i0_v, i1_v, rows_v, gsem, osem):
  c = lax.axis_index("c")
  s = lax.axis_index("s")
  wid = c * NS + s
  k = KD
  worker_rows = M_PAD // 32 // 128
  row0 = wid * worker_rows
  n_pairs = worker_rows // k // 2
  GB = k * 128 * 64 * 4

  def load(ci, b):
    r = row0 + ci * k
    pltpu.sync_copy(ud0_hbm.at[pl.ds(r, k)], i0_v.at[b])
    pltpu.sync_copy(ud1_hbm.at[pl.ds(r, k)], i1_v.at[b])

  def g1(b):
    for j in range(k):
      pltpu.async_copy(x_hbm.at[i0_v.at[b, j]],
                       rows_v.at[b, pl.ds(j * 128, 128)], gsem.at[b, j])

  def g2(b):
    for j in range(k):
      pltpu.make_async_copy(x_hbm.at[i0_v.at[b, j]],
                            rows_v.at[b, pl.ds(j * 128, 128)],
                            gsem.at[b, j]).wait()
      pltpu.async_copy(x_hbm.at[i1_v.at[b, j]],
                       rows_v.at[b, pl.ds(j * 128, 128)], gsem.at[b, j],
                       add=True)

  def store(ci, b):
    pltpu.async_copy(rows_v.at[b],
                     out_hbm.at[pl.ds((row0 + ci * k) * 128, k * 128)],
                     osem.at[b])

  def drain_g2(b):
    for j in range(k):
      pltpu.make_async_copy(x_hbm.at[i1_v.at[b, j]],
                            rows_v.at[b, pl.ds(j * 128, 128)],
                            gsem.at[b, j]).wait()

  def drain_o(b):
    pltpu.make_async_copy(rows_v.at[b], out_hbm.at[pl.ds(0, k * 128)],
                          osem.at[b]).wait()

  load(0, 0)
  g1(0)

  def pair(p, carry):
    @pl.when(p > 0)
    def _():
      drain_o(1)
    load(2 * p + 1, 1)
    g1(1)
    g2(0)
    drain_g2(0)
    store(2 * p, 0)
    @pl.when(p < n_pairs - 1)
    def _():
      drain_o(0)
      load(2 * p + 2, 0)
      g1(0)
    g2(1)
    drain_g2(1)
    store(2 * p + 1, 1)
    return carry

  lax.fori_loop(0, n_pairs, pair, 0)
  drain_o(0)
  drain_o(1)


_R = 2048


def _dinv_of(deg_ref):
  d = deg_ref[0, :] + deg_ref[1, :]
  return jnp.where(d > 0, lax.rsqrt(d), 0.0)


def _tc_prescale_body(deg_ref, x_ref, y_ref):
  dinv = _dinv_of(deg_ref)
  y_ref[...] = x_ref[...] * dinv[:, None]


def _tc_conv1_body(deg_ref, agg_ref, w_ref, b_ref, x1_ref, y2_ref):
  dinv = _dinv_of(deg_ref)
  agg = (agg_ref[:, :8] + agg_ref[:, 8:16]) * dinv[:, None]
  h = jnp.dot(agg, w_ref[...], preferred_element_type=jnp.float32) + b_ref[...]
  x1 = _leaky(h)
  x1_ref[...] = x1
  y2 = x1 * dinv[:, None]
  y2_ref[0] = y2[:, :16]
  y2_ref[1] = y2[:, 16:]


def _tc_conv2_body(deg_ref, agg_ref, x1_ref, w_ref, b_ref, x2_ref, y3_ref):
  dinv = _dinv_of(deg_ref)
  agg = agg_ref[:, :32] * dinv[:, None]
  h = jnp.dot(agg, w_ref[...], preferred_element_type=jnp.float32) + b_ref[...]
  x2 = _leaky(h) + x1_ref[...]
  x2_ref[...] = x2
  y3 = x2 * dinv[:, None]
  y3_ref[0] = y3[:, :16]
  y3_ref[1] = y3[:, 16:]


def _tc_conv3_body(deg_ref, agg_ref, w_ref, b_ref, x3_ref):
  dinv = _dinv_of(deg_ref)
  agg = agg_ref[:, :32] * dinv[:, None]
  x3_ref[...] = (
      jnp.dot(agg, w_ref[...], preferred_element_type=jnp.float32) + b_ref[...])


def _mlp3(h, w0, b0, w1, b1, w2, b2):
  h = _leaky(jnp.dot(h, w0, preferred_element_type=jnp.float32) + b0)
  h = _leaky(jnp.dot(h, w1, preferred_element_type=jnp.float32) + b1)
  return jnp.dot(h, w2, preferred_element_type=jnp.float32) + b2


def _tc_heads_body(x_ref, w0, b0, w1, b1, w2, b2, o_ref):
  o_ref[...] = _mlp3(x_ref[...], w0[...], b0[...], w1[...], b1[...],
                     w2[...], b2[...])


def _full(shape):
  zeros = (0,) * len(shape)
  return pl.BlockSpec(shape, lambda i, z=zeros: z)


def _rows(shape, dim=0):
  def imap(i):
    idx = [0] * len(shape)
    idx[dim] = i
    return tuple(idx)
  return pl.BlockSpec(shape, imap)


def _blockdiag(a, b):
  za = jnp.zeros((a.shape[0], b.shape[1]), a.dtype)
  zb = jnp.zeros((b.shape[0], a.shape[1]), a.dtype)
  return jnp.concatenate([jnp.concatenate([a, za], 1),
                          jnp.concatenate([zb, b], 1)], 0)


def _run_heads(xpacked, pairs_total, rblock, p, n1, n2, d1, d2):
  dsum = d1 + d2
  w0c = jnp.concatenate([p[f'{n1}_w0'], p[f'{n2}_w0']], axis=1)
  b0c = jnp.concatenate([p[f'{n1}_b0'], p[f'{n2}_b0']])
  w1c = _blockdiag(p[f'{n1}_w1'], p[f'{n2}_w1'])
  b1c = jnp.concatenate([p[f'{n1}_b1'], p[f'{n2}_b1']])
  w2c = _blockdiag(p[f'{n1}_w2'], p[f'{n2}_w2'])
  b2c = jnp.concatenate([p[f'{n1}_b2'], p[f'{n2}_b2']])
  w0 = _blockdiag(w0c, w0c)
  b0 = jnp.tile(b0c, 2).reshape(1, 256)
  w1 = _blockdiag(w1c, w1c)
  b1 = jnp.tile(b1c, 2).reshape(1, 256)
  w2 = jnp.zeros((256, 128), jnp.float32)
  w2 = w2.at[:128, :dsum].set(w2c).at[128:, 64:64 + dsum].set(w2c)
  b2 = jnp.zeros((128,), jnp.float32)
  b2 = b2.at[:dsum].set(b2c).at[64:64 + dsum].set(b2c).reshape(1, 128)
  wargs = [w0, b0, w1, b1, w2, b2]
  wspecs = [_full(w.shape) for w in wargs]
  out = pl.pallas_call(
      _tc_heads_body,
      grid=(pairs_total // rblock,),
      in_specs=[_rows((rblock, 128))] + wspecs,
      out_specs=_rows((rblock, 128)),
      out_shape=jax.ShapeDtypeStruct((pairs_total, 128), jnp.float32),
  )(xpacked, *wargs)
  t = out.reshape(pairs_total, 2, 64)
  oa = t[:, :, :d1].reshape(2 * pairs_total, d1)
  ob = t[:, :, d1:d1 + d2].reshape(2 * pairs_total, d2)
  return oa, ob


def kernel(x, edge_index, ud_edges, params):
  p = params
  f32 = jnp.float32

  dummy = lambda n: (jnp.arange(n, dtype=jnp.int32) % PAD_ROWS) + N
  src = jnp.concatenate([edge_index[0].astype(jnp.int32),
                         dummy(E_PAD - edge_index.shape[1])])
  dst = jnp.concatenate([edge_index[1].astype(jnp.int32),
                         dummy(E_PAD - edge_index.shape[1])])
  src2 = src.reshape(E_PAD // 128, 128)
  dst2 = dst.reshape(E_PAD // 128, 128)
  m = ud_edges.shape[0]
  ud0 = jnp.concatenate([ud_edges[:, 0].astype(jnp.int32), dummy(M_PAD - m)])
  ud1 = jnp.concatenate([ud_edges[:, 1].astype(jnp.int32), dummy(M_PAD - m)])
  ud0_2 = ud0.reshape(M_PAD // 128, 128)
  ud1_2 = ud1.reshape(M_PAD // 128, 128)
  x8 = jnp.pad(x[:, :6].astype(f32), ((0, N_PAD - N), (0, 2)))
  w0p = jnp.pad(p['gcn_w0'], ((0, 2), (0, 0)))
  z1 = jnp.zeros((ROWS_PER_SUB,), f32)
  z8 = jnp.zeros((ROWS_PER_SUB, 8), f32)
  z16 = jnp.zeros((ROWS_PER_SUB, 16), f32)

  deg2 = _sc_deg()(dst2, z1)

  y1 = pl.pallas_call(
      _tc_prescale_body,
      grid=(N_PAD // _R,),
      in_specs=[_rows((2, _R), dim=1), _rows((_R, 8))],
      out_specs=_rows((_R, 8)),
      out_shape=jax.ShapeDtypeStruct((N_PAD, 8), f32),
  )(deg2, x8)
  agg1 = _make_agg_edge_split(8)(src2, dst2, y1, z8)
  x1, y2 = pl.pallas_call(
      _tc_conv1_body,
      grid=(N_PAD // _R,),
      in_specs=[_rows((2, _R), dim=1), _rows((_R, 128)),
                _full((8, 32)), _full((1, 32))],
      out_specs=[_rows((_R, 32)), _rows((2, _R, 16), dim=1)],
      out_shape=[jax.ShapeDtypeStruct((N_PAD, 32), f32),
                 jax.ShapeDtypeStruct((2, N_PAD, 16), f32)],
  )(deg2, agg1, w0p, p['gcn_b0'].reshape(1, 32))

  agg2 = _sc_agg_feat_split()(src2, dst2, y2.reshape(2 * N_PAD, 16), z16)
  x2, y3 = pl.pallas_call(
      _tc_conv2_body,
      grid=(N_PAD // _R,),
      in_specs=[_rows((2, _R), dim=1), _rows((_R, 128)),
                _rows((_R, 32)), _full((32, 32)), _full((1, 32))],
      out_specs=[_rows((_R, 32)), _rows((2, _R, 16), dim=1)],
      out_shape=[jax.ShapeDtypeStruct((N_PAD, 32), f32),
                 jax.ShapeDtypeStruct((2, N_PAD, 16), f32)],
  )(deg2, agg2, x1, p['gcn_w1'], p['gcn_b1'].reshape(1, 32))

  agg3 = _sc_agg_feat_split()(src2, dst2, y3.reshape(2 * N_PAD, 16), z16)
  x3 = pl.pallas_call(
      _tc_conv3_body,
      grid=(N_PAD // _R,),
      in_specs=[_rows((2, _R), dim=1), _rows((_R, 128)),
                _full((32, 64)), _full((1, 64))],
      out_specs=_rows((_R, 64)),
      out_shape=jax.ShapeDtypeStruct((N_PAD, 64), f32),
  )(deg2, agg3, p['gcn_w2'], p['gcn_b2'].reshape(1, 64))

  ef = _sc_edge_feat()(ud0_2, ud1_2, x3)

  vp, vd = _run_heads(x3.reshape(N_PAD // 2, 128), N // 2, 1000,
                      p, 'pn', 'dn', 10, 10)
  ep, ed = _run_heads(ef.reshape(M_PAD // 2, 128), m // 2, 1000,
                      p, 'pe', 'de', 10, 16)

  return (x3[:N], vp, vd, ep, ed)


# speedup vs baseline: 17.7496x; 1.0404x over previous
import functools

import jax
import jax.numpy as jnp
from jax import lax
from jax.experimental import pallas as pl
from jax.experimental.pallas import tpu as pltpu
from jax.experimental.pallas import tpu_sc as plsc

N = 100000
N_PAD = 100352
PAD_ROWS = N_PAD - N
E_PAD = 1605632
M_PAD = 819200
NC, NS = 2, 16
K = 8
CHUNK = K * 128
ROWS_PER_SUB = N_PAD // NS
SLOPE = 0.1

@functools.lru_cache(maxsize=None)
def _mesh():
  return plsc.VectorSubcoreMesh(
      core_axis_name="c", subcore_axis_name="s", num_cores=NC, num_subcores=NS)


def _leaky(h):
  return jnp.where(h > 0, h, SLOPE * h)


KD = 4


@functools.lru_cache(maxsize=None)
def _sc_deg():
  return pl.kernel(
      _sc_deg_body,
      out_type=jax.ShapeDtypeStruct((NC, N_PAD), jnp.float32),
      mesh=_mesh(),
      compiler_params=pltpu.CompilerParams(use_tc_tiling_on_sc=False),
      scratch_types=[
          pltpu.VMEM((2, KD, 128), jnp.int32),
          pltpu.VMEM((128,), jnp.float32),
          pltpu.VMEM_SHARED((N_PAD,), jnp.float32),
          pltpu.SemaphoreType.DMA((2,)),
      ],
  )


def _sc_deg_body(dst2_hbm, zeros1_hbm, out_hbm, dst_v, ones_v, acc, ssem):
  c = lax.axis_index("c")
  s = lax.axis_index("s")
  wid = c * NS + s
  pltpu.sync_copy(zeros1_hbm, acc.at[pl.ds(s * ROWS_PER_SUB, ROWS_PER_SUB)])
  for i in range(8):
    ones_v[pl.ds(i * 16, 16)] = jnp.ones((16,), jnp.float32)
  plsc.subcore_barrier()
  worker_rows = E_PAD // 32 // 128
  row0 = wid * worker_rows
  n_pairs = worker_rows // KD // 2
  SB = KD * 128 * 4

  def load(ci, b):
    pltpu.sync_copy(dst2_hbm.at[pl.ds(row0 + ci * KD, KD)], dst_v.at[b])

  def issue_s(b):
    for j in range(KD):
      pltpu.async_copy(ones_v, acc.at[dst_v.at[b, j]], ssem.at[b], add=True)

  def drain_s(b):
    for j in range(KD):
      pltpu.make_async_copy(ones_v, acc.at[dst_v.at[b, j]], ssem.at[b]).wait()

  load(0, 0)

  def pair(p, carry):
    issue_s(0)
    @pl.when(p > 0)
    def _():
      drain_s(1)
    load(2 * p + 1, 1)
    issue_s(1)
    @pl.when(p < n_pairs - 1)
    def _():
      drain_s(0)
      load(2 * p + 2, 0)
    return carry

  lax.fori_loop(0, n_pairs, pair, 0)
  drain_s(0)
  drain_s(1)
  plsc.subcore_barrier()
  sl = pl.ds(s * ROWS_PER_SUB, ROWS_PER_SUB)
  pltpu.sync_copy(acc.at[sl], out_hbm.at[c, sl])


def _agg_body(src2_hbm, dst2_hbm, ytab_hbm, zeros_hbm, out_hbm,
              src_v, dst_v, rows_v, acc, gsem, ssem, *, width, k, feat_split):
  c = lax.axis_index("c")
  s = lax.axis_index("s")
  pltpu.sync_copy(zeros_hbm, acc.at[pl.ds(s * ROWS_PER_SUB, ROWS_PER_SUB)])
  plsc.subcore_barrier()
  if feat_split:
    worker_rows = E_PAD // NS // 128
    row0 = s * worker_rows
    bias = c * N_PAD
  else:
    worker_rows = E_PAD // 32 // 128
    row0 = (c * NS + s) * worker_rows
  n_chunks = worker_rows // k
  n_trip = n_chunks // 3
  n_tail = n_chunks - 3 * n_trip

  def load(ci, b):
    r = row0 + ci * k
    pltpu.sync_copy(src2_hbm.at[pl.ds(r, k)], src_v.at[b])
    pltpu.sync_copy(dst2_hbm.at[pl.ds(r, k)], dst_v.at[b])
    if feat_split:
      for j in range(k):
        for t in range(8):
          sl = (b, j, pl.ds(t * 16, 16))
          src_v[sl] = src_v[sl] + bias

  def issue_g(b):
    for j in range(k):
      pltpu.async_copy(ytab_hbm.at[src_v.at[b, j]],
                       rows_v.at[b, pl.ds(j * 128, 128)], gsem.at[b])

  def issue_s(b):
    for j in range(k):
      pltpu.async_copy(rows_v.at[b, pl.ds(j * 128, 128)],
                       acc.at[dst_v.at[b, j]], ssem.at[b], add=True)

  def drain_g(b):
    for j in range(k):
      pltpu.make_async_copy(ytab_hbm.at[src_v.at[b, j]],
                            rows_v.at[b, pl.ds(j * 128, 128)],
                            gsem.at[b]).wait()

  def drain_s(b):
    for j in range(k):
      pltpu.make_async_copy(rows_v.at[b, pl.ds(j * 128, 128)],
                            acc.at[dst_v.at[b, j]], ssem.at[b]).wait()

  load(0, 0)
  issue_g(0)
  load(1, 1)
  issue_g(1)

  def triple(t, carry):
    for b in range(3):
      ch = 3 * t + b
      drain_g(b)
      issue_s(b)
      if b == 0:
        @pl.when(t > 0)
        def _():
          drain_s(2)
      else:
        drain_s((b + 2) % 3)
      lim = n_chunks - 2 - b
      thr = (lim + 2) // 3
      nxt = (b + 2) % 3
      if thr >= n_trip:
        load(ch + 2, nxt)
        issue_g(nxt)
      else:
        @pl.when(t < thr)
        def _():
          load(ch + 2, nxt)
          issue_g(nxt)
    return carry

  lax.fori_loop(0, n_trip, triple, 0)
  for i in range(n_tail):
    ch = 3 * n_trip + i
    drain_g(i)
    issue_s(i)
    drain_s((i + 2) % 3)
  drain_s((n_chunks - 1) % 3)
  plsc.subcore_barrier()
  sl = pl.ds(s * ROWS_PER_SUB, ROWS_PER_SUB)
  pltpu.sync_copy(acc.at[sl], out_hbm.at[sl, pl.ds(c * width, width)])


@functools.lru_cache(maxsize=None)
def _make_agg_edge_split(width):
  k = KD
  body = functools.partial(_agg_body, width=width, k=k, feat_split=False)
  return pl.kernel(
      body,
      out_type=jax.ShapeDtypeStruct((N_PAD, 128), jnp.float32),
      mesh=_mesh(),
      compiler_params=pltpu.CompilerParams(use_tc_tiling_on_sc=False),
      scratch_types=[
          pltpu.VMEM((3, k, 128), jnp.int32),
          pltpu.VMEM((3, k, 128), jnp.int32),
          pltpu.VMEM((3, k * 128, width), jnp.float32),
          pltpu.VMEM_SHARED((N_PAD, width), jnp.float32),
          pltpu.SemaphoreType.DMA((3,)),
          pltpu.SemaphoreType.DMA((3,)),
      ],
  )


@functools.lru_cache(maxsize=None)
def _sc_agg_feat_split():
  k = KD
  body = functools.partial(_agg_body, width=16, k=k, feat_split=True)
  return pl.kernel(
      body,
      out_type=jax.ShapeDtypeStruct((N_PAD, 128), jnp.float32),
      mesh=_mesh(),
      compiler_params=pltpu.CompilerParams(use_tc_tiling_on_sc=False),
      scratch_types=[
          pltpu.VMEM((3, k, 128), jnp.int32),
          pltpu.VMEM((3, k, 128), jnp.int32),
          pltpu.VMEM((3, k * 128, 16), jnp.float32),
          pltpu.VMEM_SHARED((N_PAD, 16), jnp.float32),
          pltpu.SemaphoreType.DMA((3,)),
          pltpu.SemaphoreType.DMA((3,)),
      ],
  )


@functools.lru_cache(maxsize=None)
def _sc_edge_feat():
  return pl.kernel(
      _sc_edge_feat_body,
      out_type=jax.ShapeDtypeStruct((M_PAD, 64), jnp.float32),
      mesh=_mesh(),
      compiler_params=pltpu.CompilerParams(use_tc_tiling_on_sc=False),
      scratch_types=[
          pltpu.VMEM((2, KD, 128), jnp.int32),
          pltpu.VMEM((2, KD, 128), jnp.int32),
          pltpu.VMEM((2, KD * 128, 64), jnp.float32),
          pltpu.SemaphoreType.DMA((2, KD)),
          pltpu.SemaphoreType.DMA((2,)),
      ],
  )


def _sc_edge_feat_body(ud0_hbm, ud1_hbm, x_hbm, out_hbm,
                       i0_v, i1_v, rows_v, gsem, osem):
  c = lax.axis_index("c")
  s = lax.axis_index("s")
  wid = c * NS + s
  k = KD
  worker_rows = M_PAD // 32 // 128
  row0 = wid * worker_rows
  n_pairs = worker_rows // k // 2
  GB = k * 128 * 64 * 4

  def load(ci, b):
    r = row0 + ci * k
    pltpu.sync_copy(ud0_hbm.at[pl.ds(r, k)], i0_v.at[b])
    pltpu.sync_copy(ud1_hbm.at[pl.ds(r, k)], i1_v.at[b])

  def g1(b):
    for j in range(k):
      pltpu.async_copy(x_hbm.at[i0_v.at[b, j]],
                       rows_v.at[b, pl.ds(j * 128, 128)], gsem.at[b, j])

  def g2(b):
    for j in range(k):
      pltpu.make_async_copy(x_hbm.at[i0_v.at[b, j]],
                            rows_v.at[b, pl.ds(j * 128, 128)],
                            gsem.at[b, j]).wait()
      pltpu.async_copy(x_hbm.at[i1_v.at[b, j]],
                       rows_v.at[b, pl.ds(j * 128, 128)], gsem.at[b, j],
                       add=True)

  def store(ci, b):
    pltpu.async_copy(rows_v.at[b],
                     out_hbm.at[pl.ds((row0 + ci * k) * 128, k * 128)],
                     osem.at[b])

  def drain_g2(b):
    for j in range(k):
      pltpu.make_async_copy(x_hbm.at[i1_v.at[b, j]],
                            rows_v.at[b, pl.ds(j * 128, 128)],
                            gsem.at[b, j]).wait()

  def drain_o(b):
    pltpu.make_async_copy(rows_v.at[b], out_hbm.at[pl.ds(0, k * 128)],
                          osem.at[b]).wait()

  load(0, 0)
  g1(0)

  def pair(p, carry):
    @pl.when(p > 0)
    def _():
      drain_o(1)
    load(2 * p + 1, 1)
    g1(1)
    g2(0)
    drain_g2(0)
    store(2 * p, 0)
    @pl.when(p < n_pairs - 1)
    def _():
      drain_o(0)
      load(2 * p + 2, 0)
      g1(0)
    g2(1)
    drain_g2(1)
    store(2 * p + 1, 1)
    return carry

  lax.fori_loop(0, n_pairs, pair, 0)
  drain_o(0)
  drain_o(1)


_R = 2048


def _dinv_of(deg_ref):
  d = deg_ref[0, :] + deg_ref[1, :]
  return jnp.where(d > 0, lax.rsqrt(d), 0.0)


def _tc_prescale_body(deg_ref, x_ref, y_ref):
  dinv = _dinv_of(deg_ref)
  y_ref[...] = x_ref[...] * dinv[:, None]


def _tc_conv1_body(deg_ref, agg_ref, w_ref, b_ref, x1_ref, y2_ref):
  dinv = _dinv_of(deg_ref)
  agg = (agg_ref[:, :8] + agg_ref[:, 8:16]) * dinv[:, None]
  h = jnp.dot(agg, w_ref[...], preferred_element_type=jnp.float32) + b_ref[...]
  x1 = _leaky(h)
  x1_ref[...] = x1
  y2 = x1 * dinv[:, None]
  y2_ref[0] = y2[:, :16]
  y2_ref[1] = y2[:, 16:]


def _tc_conv2_body(deg_ref, agg_ref, x1_ref, w_ref, b_ref, x2_ref, y3_ref):
  dinv = _dinv_of(deg_ref)
  agg = agg_ref[:, :32] * dinv[:, None]
  h = jnp.dot(agg, w_ref[...], preferred_element_type=jnp.float32) + b_ref[...]
  x2 = _leaky(h) + x1_ref[...]
  x2_ref[...] = x2
  y3 = x2 * dinv[:, None]
  y3_ref[0] = y3[:, :16]
  y3_ref[1] = y3[:, 16:]


def _tc_conv3_body(deg_ref, agg_ref, w_ref, b_ref, x3_ref):
  dinv = _dinv_of(deg_ref)
  agg = agg_ref[:, :32] * dinv[:, None]
  x3_ref[...] = (
      jnp.dot(agg, w_ref[...], preferred_element_type=jnp.float32) + b_ref[...])


def _mlp3(h, w0, b0, w1, b1, w2, b2):
  h = _leaky(jnp.dot(h, w0, preferred_element_type=jnp.float32) + b0)
  h = _leaky(jnp.dot(h, w1, preferred_element_type=jnp.float32) + b1)
  return jnp.dot(h, w2, preferred_element_type=jnp.float32) + b2


def _tc_heads_body(x_ref, w0, b0, w1, b1, w2, b2, o_ref):
  o_ref[...] = _mlp3(x_ref[...], w0[...], b0[...], w1[...], b1[...],
                     w2[...], b2[...])


def _full(shape):
  zeros = (0,) * len(shape)
  return pl.BlockSpec(shape, lambda i, z=zeros: z)


def _rows(shape, dim=0):
  def imap(i):
    idx = [0] * len(shape)
    idx[dim] = i
    return tuple(idx)
  return pl.BlockSpec(shape, imap)


def _blockdiag(a, b):
  za = jnp.zeros((a.shape[0], b.shape[1]), a.dtype)
  zb = jnp.zeros((b.shape[0], a.shape[1]), a.dtype)
  return jnp.concatenate([jnp.concatenate([a, za], 1),
                          jnp.concatenate([zb, b], 1)], 0)


def _run_heads(xpacked, pairs_total, rblock, p, n1, n2, d1, d2):
  dsum = d1 + d2
  w0c = jnp.concatenate([p[f'{n1}_w0'], p[f'{n2}_w0']], axis=1)
  b0c = jnp.concatenate([p[f'{n1}_b0'], p[f'{n2}_b0']])
  w1c = _blockdiag(p[f'{n1}_w1'], p[f'{n2}_w1'])
  b1c = jnp.concatenate([p[f'{n1}_b1'], p[f'{n2}_b1']])
  w2c = _blockdiag(p[f'{n1}_w2'], p[f'{n2}_w2'])
  b2c = jnp.concatenate([p[f'{n1}_b2'], p[f'{n2}_b2']])
  w0 = _blockdiag(w0c, w0c)
  b0 = jnp.tile(b0c, 2).reshape(1, 256)
  w1 = _blockdiag(w1c, w1c)
  b1 = jnp.tile(b1c, 2).reshape(1, 256)
  w2 = jnp.zeros((256, 128), jnp.float32)
  w2 = w2.at[:128, :dsum].set(w2c).at[128:, 64:64 + dsum].set(w2c)
  b2 = jnp.zeros((128,), jnp.float32)
  b2 = b2.at[:dsum].set(b2c).at[64:64 + dsum].set(b2c).reshape(1, 128)
  wargs = [w0, b0, w1, b1, w2, b2]
  wspecs = [_full(w.shape) for w in wargs]
  out = pl.pallas_call(
      _tc_heads_body,
      grid=(pairs_total // rblock,),
      in_specs=[_rows((rblock, 128))] + wspecs,
      out_specs=_rows((rblock, 128)),
      out_shape=jax.ShapeDtypeStruct((pairs_total, 128), jnp.float32),
  )(xpacked, *wargs)
  t = out.reshape(pairs_total, 2, 64)
  oa = t[:, :, :d1].reshape(2 * pairs_total, d1)
  ob = t[:, :, d1:d1 + d2].reshape(2 * pairs_total, d2)
  return oa, ob


def kernel(x, edge_index, ud_edges, params):
  p = params
  f32 = jnp.float32

  dummy = lambda n: (jnp.arange(n, dtype=jnp.int32) % PAD_ROWS) + N
  src = jnp.concatenate([edge_index[0].astype(jnp.int32),
                         dummy(E_PAD - edge_index.shape[1])])
  dst = jnp.concatenate([edge_index[1].astype(jnp.int32),
                         dummy(E_PAD - edge_index.shape[1])])
  src2 = src.reshape(E_PAD // 128, 128)
  dst2 = dst.reshape(E_PAD // 128, 128)
  m = ud_edges.shape[0]
  ud0 = jnp.concatenate([ud_edges[:, 0].astype(jnp.int32), dummy(M_PAD - m)])
  ud1 = jnp.concatenate([ud_edges[:, 1].astype(jnp.int32), dummy(M_PAD - m)])
  ud0_2 = ud0.reshape(M_PAD // 128, 128)
  ud1_2 = ud1.reshape(M_PAD // 128, 128)
  x8 = jnp.pad(x[:, :6].astype(f32), ((0, N_PAD - N), (0, 2)))
  w0p = jnp.pad(p['gcn_w0'], ((0, 2), (0, 0)))
  z1 = jnp.zeros((ROWS_PER_SUB,), f32)
  z8 = jnp.zeros((ROWS_PER_SUB, 8), f32)
  z16 = jnp.zeros((ROWS_PER_SUB, 16), f32)

  deg2 = _sc_deg()(dst2, z1)

  y1 = pl.pallas_call(
      _tc_prescale_body,
      grid=(N_PAD // _R,),
      in_specs=[_rows((2, _R), dim=1), _rows((_R, 8))],
      out_specs=_rows((_R, 8)),
      out_shape=jax.ShapeDtypeStruct((N_PAD, 8), f32),
  )(deg2, x8)
  agg1 = _make_agg_edge_split(8)(src2, dst2, y1, z8)
  x1, y2 = pl.pallas_call(
      _tc_conv1_body,
      grid=(N_PAD // _R,),
      in_specs=[_rows((2, _R), dim=1), _rows((_R, 128)),
                _full((8, 32)), _full((1, 32))],
      out_specs=[_rows((_R, 32)), _rows((2, _R, 16), dim=1)],
      out_shape=[jax.ShapeDtypeStruct((N_PAD, 32), f32),
                 jax.ShapeDtypeStruct((2, N_PAD, 16), f32)],
  )(deg2, agg1, w0p, p['gcn_b0'].reshape(1, 32))

  agg2 = _sc_agg_feat_split()(src2, dst2, y2.reshape(2 * N_PAD, 16), z16)
  x2, y3 = pl.pallas_call(
      _tc_conv2_body,
      grid=(N_PAD // _R,),
      in_specs=[_rows((2, _R), dim=1), _rows((_R, 128)),
                _rows((_R, 32)), _full((32, 32)), _full((1, 32))],
      out_specs=[_rows((_R, 32)), _rows((2, _R, 16), dim=1)],
      out_shape=[jax.ShapeDtypeStruct((N_PAD, 32), f32),
                 jax.ShapeDtypeStruct((2, N_PAD, 16), f32)],
  )(deg2, agg2, x1, p['gcn_w1'], p['gcn_b1'].reshape(1, 32))

  agg3 = _sc_agg_feat_split()(src2, dst2, y3.reshape(2 * N_PAD, 16), z16)
  x3 = pl.pallas_call(
      _tc_conv3_body,
      grid=(N_PAD // _R,),
      in_specs=[_rows((2, _R), dim=1), _rows((_R, 128)),
                _full((32, 64)), _full((1, 64))],
      out_specs=_rows((_R, 64)),
      out_shape=jax.ShapeDtypeStruct((N_PAD, 64), f32),
  )(deg2, agg3, p['gcn_w2'], p['gcn_b2'].reshape(1, 64))

  ef = _sc_edge_feat()(ud0_2, ud1_2, x3)

  vp, vd = _run_heads(x3.reshape(N_PAD // 2, 128), N // 2, 1000,
                      p, 'pn', 'dn', 10, 10)
  ep, ed = _run_heads(ef.reshape(M_PAD // 2, 128), m // 2, 1000,
                      p, 'pe', 'de', 10, 16)

  return (x3[:N], vp, vd, ep, ed)
